# Initial kernel scaffold; baseline (speedup 1.0000x reference)
import jax
import jax.numpy as jnp
from jax.experimental import pallas as pl


def kernel(input_ids, hidden_states, loss_mask, embed_table, Wq, Wk, Wv, Wo, W1, W2, Wlm):
    raise NotImplementedError("write your pallas kernel here")


# speedup vs baseline: 1.2985x; 1.2985x over previous
import functools

import numpy as np
import jax
import jax.numpy as jnp
from jax import lax
from jax.experimental import pallas as pl
from jax.experimental.pallas import tpu as pltpu
from jax.experimental.pallas import tpu_sc as plsc

SEQ = 4096
D = 1024
H = 16
DH = 64
HALF = DH // 2
PD = 128
HPD = H * PD
VOCAB = 32000
BS = 16
NA = 128
DFF = 2048
QL = NA * BS
KL = NA + QL
KB = 1 + BS
VT = 1280
NVT = VOCAB // VT

_F32 = jnp.float32
_I32 = jnp.int32


def _np_mask_const():
    r = np.arange(BS * H)[:, None]
    c = np.arange(KB * H)[None, :]
    return (r % H == c % H).astype(np.float32)


def _np_g_const():
    c = np.arange(KB * H)[:, None]
    e = np.arange(32)[None, :]
    return (c // H == e).astype(np.float32)


_MASK_HH = _np_mask_const()
_G_COLLAPSE = _np_g_const()
_GT_EXPAND = _G_COLLAPSE.T.copy()


def _rot_cols(w):
    w3 = w.reshape(D, H, 2, HALF)
    return jnp.stack([-w3[:, :, 1], w3[:, :, 0]], axis=2).reshape(D, D)


def _pad_heads(w):
    w3 = w.reshape(D, H, DH)
    return jnp.pad(w3, ((0, 0), (0, 0), (0, PD - DH))).reshape(D, HPD)


def _pad_head_rows(w):
    w3 = w.reshape(H, DH, D)
    return jnp.pad(w3, ((0, 0), (0, PD - DH), (0, 0))).reshape(HPD, D)


def _sc_gather_stage(anchors, keep_i, ids, lm, embed_table, hs):
    mesh = plsc.VectorSubcoreMesh(core_axis_name="c", subcore_axis_name="s")

    @functools.partial(
        pl.kernel,
        mesh=mesh,
        compiler_params=pltpu.CompilerParams(needs_layout_passes=False),
        out_type=[
            jax.ShapeDtypeStruct((QL, D), _F32),
            jax.ShapeDtypeStruct((NA, D), _F32),
            jax.ShapeDtypeStruct((QL,), _I32),
            jax.ShapeDtypeStruct((QL,), _F32),
        ],
        scratch_types=[
            pltpu.VMEM((NA,), _I32),
            pltpu.VMEM((NA,), _I32),
            pltpu.VMEM((SEQ,), _I32),
            pltpu.VMEM((SEQ,), _F32),
            pltpu.VMEM((4 * BS,), _I32),
            pltpu.VMEM((16,), _I32),
            pltpu.VMEM((4 * BS, D), _F32),
            pltpu.VMEM((16, D), _F32),
            pltpu.VMEM((4 * BS,), _I32),
            pltpu.VMEM((4 * BS,), _F32),
            pltpu.SemaphoreType.DMA,
        ],
    )
    def sc_kernel(anchors_hbm, keep_hbm, ids_hbm, lm_hbm, embed_hbm, hs_hbm,
                  ne_hbm, th_hbm, tgt_hbm, wlm_hbm,
                  anchors_v, keep_v, ids_v, lm_v, eidx, cidx,
                  ne_rows, hs_rows, tvec, wvec, sem):
        wid = lax.axis_index("s") * 2 + lax.axis_index("c")
        lanes = lax.iota(_I32, 16)

        pltpu.sync_copy(anchors_hbm, anchors_v)
        pltpu.sync_copy(keep_hbm, keep_v)
        pltpu.sync_copy(ids_hbm, ids_v)
        pltpu.sync_copy(lm_hbm, lm_v)

        b0 = wid * 4
        am = lanes < 4
        aidx = jnp.minimum(b0 + lanes, NA - 1)
        a_v = plsc.load_gather(anchors_v, [aidx])
        k_v = plsc.load_gather(keep_v, [aidx])
        tok_v = plsc.load_gather(ids_v, [jnp.clip(a_v, 0, SEQ - 1)])
        tok_v = jnp.where((k_v > 0) & am, tok_v, 0)
        for j in range(4):
            eidx[pl.ds(16 * j, 16)] = jnp.zeros((16,), _I32)
        plsc.store_scatter(eidx, [lanes * 16], tok_v, mask=am)
        pltpu.async_copy(embed_hbm.at[eidx], ne_rows, sem).wait()
        pltpu.sync_copy(ne_rows, ne_hbm.at[pl.ds(64 * wid, 64)])

        for j in range(4):
            bb = jnp.full((16,), b0 + j, _I32)
            a_b = plsc.load_gather(anchors_v, [bb])
            lidx = jnp.clip(a_b + lanes, 0, SEQ - 1)
            tvec[pl.ds(16 * j, 16)] = plsc.load_gather(ids_v, [lidx])
            wvec[pl.ds(16 * j, 16)] = plsc.load_gather(lm_v, [lidx])
        pltpu.sync_copy(tvec, tgt_hbm.at[pl.ds(64 * wid, 64)])
        pltpu.sync_copy(wvec, wlm_hbm.at[pl.ds(64 * wid, 64)])

        @pl.when(wid < 8)
        def _():
            a16 = anchors_v[pl.ds(16 * wid, 16)]
            cidx[...] = jnp.maximum(a16 - 1, 0)
            pltpu.async_copy(hs_hbm.at[cidx], hs_rows, sem).wait()
            pltpu.sync_copy(hs_rows, th_hbm.at[pl.ds(16 * wid, 16)])

    return sc_kernel(anchors, keep_i, ids, lm, embed_table, hs)


def _rope_mul(a, b, pos):
    t, _ = a.shape
    col = lax.broadcasted_iota(_I32, (t, HPD), 1)
    freq = jnp.exp((col % HALF).astype(_F32) * _F32(-np.log(10000.0) / HALF))
    ang = pos * freq
    return a * jnp.cos(ang) + b * jnp.sin(ang)


def _q_proj_kernel(x_ref, pos_ref, wq_ref, wqr_ref, q_ref):
    x = x_ref[...]
    a = jnp.dot(x, wq_ref[...], preferred_element_type=_F32)
    b = jnp.dot(x, wqr_ref[...], preferred_element_type=_F32)
    q_ref[...] = _rope_mul(a, b, pos_ref[...])


def _kv_proj_kernel(x_ref, pos_ref, wk_ref, wkr_ref, wv_ref, k_ref, v_ref):
    x = x_ref[...]
    a = jnp.dot(x, wk_ref[...], preferred_element_type=_F32)
    b = jnp.dot(x, wkr_ref[...], preferred_element_type=_F32)
    k_ref[...] = _rope_mul(a, b, pos_ref[...])
    v_ref[...] = jnp.dot(x, wv_ref[...], preferred_element_type=_F32)


_ATT_BLOCKS_PER_STEP = 8


def _attention_kernel(q_ref, k_ref, v_ref, m_ref, g_ref, gt_ref, o_ref):
    m = m_ref[...]
    scale = _F32(1.0 / np.sqrt(DH))
    for b in range(_ATT_BLOCKS_PER_STEP):
        qb = q_ref[256 * b:256 * (b + 1), :]
        kb = k_ref[272 * b:272 * (b + 1), :]
        vb = v_ref[272 * b:272 * (b + 1), :]
        s = lax.dot_general(qb, kb, (((1,), (1,)), ((), ())),
                            preferred_element_type=_F32) * scale
        p = jnp.dot(s * m, g_ref[...], preferred_element_type=_F32)
        colmask = lax.broadcasted_iota(_I32, (BS * H, 32), 1) < KB
        mx = jnp.max(jnp.where(colmask, p, _F32(-1e30)), axis=1, keepdims=True)
        ex = jnp.where(colmask, jnp.exp(p - mx), _F32(0.0))
        pn = ex / jnp.sum(ex, axis=1, keepdims=True)
        pe = jnp.dot(pn, gt_ref[...], preferred_element_type=_F32)
        o_ref[256 * b:256 * (b + 1), :] = jnp.dot(
            pe * m, vb, preferred_element_type=_F32)


def _mlp_kernel(ne_ref, ctx_ref, wo_ref, w1_ref, w2_ref, hid_ref):
    h0 = ne_ref[...] + jnp.dot(ctx_ref[...], wo_ref[...],
                               preferred_element_type=_F32)
    h1 = jax.nn.gelu(jnp.dot(h0, w1_ref[...], preferred_element_type=_F32))
    hid_ref[...] = h0 + jnp.dot(h1, w2_ref[...], preferred_element_type=_F32)


def _ce_kernel(hid_ref, wlm_ref, tgt_ref, w_ref,
               loss_ref, acc_ref,
               m_s, s_s, tl_s, bv_s, bi_s):
    j = pl.program_id(0)

    @pl.when(j == 0)
    def _():
        m_s[...] = jnp.full((QL, 1), -1e30, _F32)
        s_s[...] = jnp.zeros((QL, 1), _F32)
        tl_s[...] = jnp.zeros((QL, 1), _F32)
        bv_s[...] = jnp.full((QL, 1), -1e30, _F32)
        bi_s[...] = jnp.zeros((QL, 1), _I32)

    logits = jnp.dot(hid_ref[...], wlm_ref[...], preferred_element_type=_F32)
    gcol = lax.broadcasted_iota(_I32, (QL, VT), 1) + j * VT
    t = tgt_ref[...]

    tmax = jnp.max(logits, axis=1, keepdims=True)
    mnew = jnp.maximum(m_s[...], tmax)
    srow = jnp.sum(jnp.exp(logits - mnew), axis=1, keepdims=True)
    s_s[...] = s_s[...] * jnp.exp(m_s[...] - mnew) + srow
    m_s[...] = mnew

    tl_s[...] += jnp.sum(jnp.where(gcol == t, logits, _F32(0.0)),
                         axis=1, keepdims=True)

    targ = jnp.min(jnp.where(logits == tmax, gcol, _I32(2 ** 30)),
                   axis=1, keepdims=True)
    upd = tmax > bv_s[...]
    bi_s[...] = jnp.where(upd, targ, bi_s[...])
    bv_s[...] = jnp.maximum(bv_s[...], tmax)

    @pl.when(j == NVT - 1)
    def _():
        wv = w_ref[...]
        lpt = m_s[...] + jnp.log(s_s[...]) - tl_s[...]
        loss_ref[0, 0] = jnp.sum(lpt * wv) / (jnp.sum(wv) + _F32(1e-6))
        sel = wv > _F32(0.5)
        corr = jnp.sum(jnp.where(sel & (bi_s[...] == t), _F32(1.0), _F32(0.0)))
        cnt = jnp.sum(jnp.where(sel, _F32(1.0), _F32(0.0)))
        acc_ref[0, 0] = corr / jnp.maximum(cnt, _F32(1.0))


def _sample_anchors_fast(lm):
    max_anchor = SEQ - BS
    valid = lm[:max_anchor + 1] > 0.5
    valid_count = valid.sum()
    rv = jax.random.uniform(jax.random.key(42), (1, max_anchor + 1))[0]
    rv = jnp.where(valid, rv, 2.0)
    idxs = jnp.arange(max_anchor + 1)
    masked_idx = jnp.where(valid, idxs, SEQ + 1)
    _, sel = lax.top_k(-rv, NA)
    anchors = jnp.sort(masked_idx[sel])
    keep = jnp.arange(NA) < jnp.minimum(valid_count, NA)
    anchors = jnp.where(keep, anchors, 0).astype(_I32)
    return anchors, keep


def kernel(input_ids, hidden_states, loss_mask, embed_table,
           Wq, Wk, Wv, Wo, W1, W2, Wlm):
    ids = input_ids[0].astype(_I32)
    lm = loss_mask[0].astype(_F32)
    hs = hidden_states[0]

    anchors, keep = _sample_anchors_fast(lm)
    keep_i = keep.astype(_I32)

    labels = anchors[:, None] + jnp.arange(BS, dtype=_I32)[None, :]
    draft_pos = labels.reshape(QL, 1).astype(_F32)
    ctx_pos_i = jnp.maximum(anchors - 1, 0)
    kv_pos = jnp.concatenate([ctx_pos_i.astype(_F32)[:, None], draft_pos],
                             axis=0)

    ne, th, tgt, wlm_g = _sc_gather_stage(anchors, keep_i, ids, lm,
                                          embed_table, hs)

    wq_p = _pad_heads(Wq)
    wqr_p = _pad_heads(_rot_cols(Wq))
    wk_p = _pad_heads(Wk)
    wkr_p = _pad_heads(_rot_cols(Wk))
    wv_p = _pad_heads(Wv)
    wo_p = _pad_head_rows(Wo)

    TQ = 256
    qp = pl.pallas_call(
        _q_proj_kernel,
        grid=(QL // TQ,),
        in_specs=[
            pl.BlockSpec((TQ, D), lambda i: (i, 0)),
            pl.BlockSpec((TQ, 1), lambda i: (i, 0)),
            pl.BlockSpec((D, HPD), lambda i: (0, 0)),
            pl.BlockSpec((D, HPD), lambda i: (0, 0)),
        ],
        out_specs=pl.BlockSpec((TQ, HPD), lambda i: (i, 0)),
        out_shape=jax.ShapeDtypeStruct((QL, HPD), _F32),
    )(ne, draft_pos, wq_p, wqr_p)

    kv_in = jnp.concatenate([th, ne], axis=0)
    TK = 128
    kp, vp = pl.pallas_call(
        _kv_proj_kernel,
        grid=(KL // TK,),
        in_specs=[
            pl.BlockSpec((TK, D), lambda i: (i, 0)),
            pl.BlockSpec((TK, 1), lambda i: (i, 0)),
            pl.BlockSpec((D, HPD), lambda i: (0, 0)),
            pl.BlockSpec((D, HPD), lambda i: (0, 0)),
            pl.BlockSpec((D, HPD), lambda i: (0, 0)),
        ],
        out_specs=[
            pl.BlockSpec((TK, HPD), lambda i: (i, 0)),
            pl.BlockSpec((TK, HPD), lambda i: (i, 0)),
        ],
        out_shape=[
            jax.ShapeDtypeStruct((KL, HPD), _F32),
            jax.ShapeDtypeStruct((KL, HPD), _F32),
        ],
    )(kv_in, kv_pos, wk_p, wkr_p, wv_p)

    q_r = qp.reshape(QL * H, PD)
    kc = kp[:NA].reshape(NA, 1, HPD)
    kd = kp[NA:].reshape(NA, BS, HPD)
    k_r = jnp.concatenate([kc, kd], axis=1).reshape(NA * KB * H, PD)
    vc = vp[:NA].reshape(NA, 1, HPD)
    vd = vp[NA:].reshape(NA, BS, HPD)
    v_r = jnp.concatenate([vc, vd], axis=1).reshape(NA * KB * H, PD)

    mask_hh = jnp.asarray(_MASK_HH)
    g_c = jnp.asarray(_G_COLLAPSE)
    gt_c = jnp.asarray(_GT_EXPAND)

    GSTEP = _ATT_BLOCKS_PER_STEP
    ctx_r = pl.pallas_call(
        _attention_kernel,
        grid=(NA // GSTEP,),
        in_specs=[
            pl.BlockSpec((GSTEP * 256, PD), lambda i: (i, 0)),
            pl.BlockSpec((GSTEP * 272, PD), lambda i: (i, 0)),
            pl.BlockSpec((GSTEP * 272, PD), lambda i: (i, 0)),
            pl.BlockSpec((BS * H, KB * H), lambda i: (0, 0)),
            pl.BlockSpec((KB * H, 32), lambda i: (0, 0)),
            pl.BlockSpec((32, KB * H), lambda i: (0, 0)),
        ],
        out_specs=pl.BlockSpec((GSTEP * 256, PD), lambda i: (i, 0)),
        out_shape=jax.ShapeDtypeStruct((QL * H, PD), _F32),
    )(q_r, k_r, v_r, mask_hh, g_c, gt_c)
    ctx_p = ctx_r.reshape(QL, HPD)

    TM = 256
    hid = pl.pallas_call(
        _mlp_kernel,
        grid=(QL // TM,),
        in_specs=[
            pl.BlockSpec((TM, D), lambda i: (i, 0)),
            pl.BlockSpec((TM, HPD), lambda i: (i, 0)),
            pl.BlockSpec((HPD, D), lambda i: (0, 0)),
            pl.BlockSpec((D, DFF), lambda i: (0, 0)),
            pl.BlockSpec((DFF, D), lambda i: (0, 0)),
        ],
        out_specs=pl.BlockSpec((TM, D), lambda i: (i, 0)),
        out_shape=jax.ShapeDtypeStruct((QL, D), _F32),
    )(ne, ctx_p, wo_p, W1, W2)

    valid_label = (labels < SEQ).astype(_F32)
    wgt = (keep.astype(_F32)[:, None] * valid_label
           * (jnp.arange(BS) > 0).astype(_F32)[None, :]).reshape(QL)
    wgt = (wgt * wlm_g).reshape(QL, 1)
    tgt2 = tgt.reshape(QL, 1)

    loss2, acc2 = pl.pallas_call(
        _ce_kernel,
        grid=(NVT,),
        in_specs=[
            pl.BlockSpec((QL, D), lambda j: (0, 0)),
            pl.BlockSpec((D, VT), lambda j: (0, j)),
            pl.BlockSpec((QL, 1), lambda j: (0, 0)),
            pl.BlockSpec((QL, 1), lambda j: (0, 0)),
        ],
        out_specs=[
            pl.BlockSpec(memory_space=pltpu.SMEM),
            pl.BlockSpec(memory_space=pltpu.SMEM),
        ],
        out_shape=[
            jax.ShapeDtypeStruct((1, 1), _F32),
            jax.ShapeDtypeStruct((1, 1), _F32),
        ],
        scratch_shapes=[
            pltpu.VMEM((QL, 1), _F32),
            pltpu.VMEM((QL, 1), _F32),
            pltpu.VMEM((QL, 1), _F32),
            pltpu.VMEM((QL, 1), _F32),
            pltpu.VMEM((QL, 1), _I32),
        ],
    )(hid, Wlm, tgt2, wgt)

    return loss2[0, 0], acc2[0, 0]


# speedup vs baseline: 1.6750x; 1.2900x over previous
import functools

import numpy as np
import jax
import jax.numpy as jnp
from jax import lax
from jax.experimental import pallas as pl
from jax.experimental.pallas import tpu as pltpu
from jax.experimental.pallas import tpu_sc as plsc

SEQ = 4096
D = 1024
H = 16
DH = 64
HALF = DH // 2
VOCAB = 32000
BS = 16
NA = 128
DFF = 2048
QL = NA * BS
KL = NA + QL
KB = 1 + BS
VT = 1280
NVT = VOCAB // VT

_F32 = jnp.float32
_I32 = jnp.int32


def _np_mask_const():
    r = np.arange(BS * H)[:, None]
    c = np.arange(KB * H)[None, :]
    return (r % H == c % H).astype(np.float32)


def _np_g_const():
    c = np.arange(KB * H)[:, None]
    e = np.arange(32)[None, :]
    return (c // H == e).astype(np.float32)


_MASK_HH = _np_mask_const()
_G_COLLAPSE = _np_g_const()
_GT_EXPAND = _G_COLLAPSE.T.copy()


def _rot_cols(w):
    w3 = w.reshape(D, H, 2, HALF)
    return jnp.stack([-w3[:, :, 1], w3[:, :, 0]], axis=2).reshape(D, D)


def _sc_gather_stage(anchors, keep_i, ids, lm, embed_table, hs):
    mesh = plsc.VectorSubcoreMesh(core_axis_name="c", subcore_axis_name="s")

    @functools.partial(
        pl.kernel,
        mesh=mesh,
        compiler_params=pltpu.CompilerParams(needs_layout_passes=False),
        out_type=[
            jax.ShapeDtypeStruct((KL, D), _F32),
            jax.ShapeDtypeStruct((QL,), _I32),
            jax.ShapeDtypeStruct((QL,), _F32),
        ],
        scratch_types=[
            pltpu.VMEM((NA,), _I32),
            pltpu.VMEM((NA,), _I32),
            pltpu.VMEM((SEQ,), _I32),
            pltpu.VMEM((SEQ,), _F32),
            pltpu.VMEM((4 * BS,), _I32),
            pltpu.VMEM((16,), _I32),
            pltpu.VMEM((4 * BS, D), _F32),
            pltpu.VMEM((16, D), _F32),
            pltpu.VMEM((4 * BS,), _I32),
            pltpu.VMEM((4 * BS,), _F32),
            pltpu.SemaphoreType.DMA,
            pltpu.SemaphoreType.DMA,
        ],
    )
    def sc_kernel(anchors_hbm, keep_hbm, ids_hbm, lm_hbm, embed_hbm, hs_hbm,
                  kvin_hbm, tgt_hbm, wlm_hbm,
                  anchors_v, keep_v, ids_v, lm_v, eidx, cidx,
                  ne_rows, hs_rows, tvec, wvec, sem, sem2):
        wid = lax.axis_index("s") * 2 + lax.axis_index("c")
        lanes = lax.iota(_I32, 16)

        c1 = pltpu.async_copy(anchors_hbm, anchors_v, sem)
        c2 = pltpu.async_copy(keep_hbm, keep_v, sem)
        c3 = pltpu.async_copy(ids_hbm, ids_v, sem)
        c4 = pltpu.async_copy(lm_hbm, lm_v, sem)
        c1.wait()
        c2.wait()
        c3.wait()
        c4.wait()

        b0 = wid * 4
        am = lanes < 4
        aidx = jnp.minimum(b0 + lanes, NA - 1)
        a_v = plsc.load_gather(anchors_v, [aidx])
        k_v = plsc.load_gather(keep_v, [aidx])
        tok_v = plsc.load_gather(ids_v, [jnp.clip(a_v, 0, SEQ - 1)])
        tok_v = jnp.where((k_v > 0) & am, tok_v, 0)
        for j in range(4):
            eidx[pl.ds(16 * j, 16)] = jnp.zeros((16,), _I32)
        plsc.store_scatter(eidx, [lanes * 16], tok_v, mask=am)
        eg = pltpu.async_copy(embed_hbm.at[eidx], ne_rows, sem2)

        for j in range(4):
            bb = jnp.full((16,), b0 + j, _I32)
            a_b = plsc.load_gather(anchors_v, [bb])
            lidx = jnp.clip(a_b + lanes, 0, SEQ - 1)
            tvec[pl.ds(16 * j, 16)] = plsc.load_gather(ids_v, [lidx])
            wvec[pl.ds(16 * j, 16)] = plsc.load_gather(lm_v, [lidx])
        t1 = pltpu.async_copy(tvec, tgt_hbm.at[pl.ds(64 * wid, 64)], sem)
        t2 = pltpu.async_copy(wvec, wlm_hbm.at[pl.ds(64 * wid, 64)], sem)

        eg.wait()
        ne_out = pltpu.async_copy(
            ne_rows, kvin_hbm.at[pl.ds(NA + 64 * wid, 64)], sem2)

        @pl.when(wid < 8)
        def _():
            a16 = anchors_v[pl.ds(16 * wid, 16)]
            cidx[...] = jnp.maximum(a16 - 1, 0)
            pltpu.async_copy(hs_hbm.at[cidx], hs_rows, sem).wait()
            pltpu.sync_copy(hs_rows, kvin_hbm.at[pl.ds(16 * wid, 16)])

        t1.wait()
        t2.wait()
        ne_out.wait()

    return sc_kernel(anchors, keep_i, ids, lm, embed_table, hs)


def _rope_mul(a, b, pos):
    t, _ = a.shape
    col = lax.broadcasted_iota(_I32, (t, D), 1)
    freq = jnp.exp((col % HALF).astype(_F32) * _F32(-np.log(10000.0) / HALF))
    ang = pos * freq
    return a * jnp.cos(ang) + b * jnp.sin(ang)


def _qkv_proj_kernel(x_ref, pos_ref, wq_ref, wqr_ref, wk_ref, wkr_ref, wv_ref,
                     q_ref, k_ref, v_ref):
    x = x_ref[...]
    pos = pos_ref[...]
    aq = jnp.dot(x, wq_ref[...], preferred_element_type=_F32)
    bq = jnp.dot(x, wqr_ref[...], preferred_element_type=_F32)
    q_ref[...] = _rope_mul(aq, bq, pos)
    ak = jnp.dot(x, wk_ref[...], preferred_element_type=_F32)
    bk = jnp.dot(x, wkr_ref[...], preferred_element_type=_F32)
    k_ref[...] = _rope_mul(ak, bk, pos)
    v_ref[...] = jnp.dot(x, wv_ref[...], preferred_element_type=_F32)


_ATT_BLOCKS_PER_STEP = 8


def _attention_kernel(q_ref, k_ref, v_ref, m_ref, g_ref, gt_ref, o_ref):
    m = m_ref[...]
    scale = _F32(1.0 / np.sqrt(DH))
    for b in range(_ATT_BLOCKS_PER_STEP):
        qb = q_ref[256 * b:256 * (b + 1), :]
        kb = k_ref[272 * b:272 * (b + 1), :]
        vb = v_ref[272 * b:272 * (b + 1), :]
        s = lax.dot_general(qb, kb, (((1,), (1,)), ((), ())),
                            preferred_element_type=_F32) * scale
        p = jnp.dot(s * m, g_ref[...], preferred_element_type=_F32)
        colmask = lax.broadcasted_iota(_I32, (BS * H, 32), 1) < KB
        mx = jnp.max(jnp.where(colmask, p, _F32(-1e30)), axis=1, keepdims=True)
        ex = jnp.where(colmask, jnp.exp(p - mx), _F32(0.0))
        pn = ex / jnp.sum(ex, axis=1, keepdims=True)
        pe = jnp.dot(pn, gt_ref[...], preferred_element_type=_F32)
        o_ref[256 * b:256 * (b + 1), :] = jnp.dot(
            pe * m, vb, preferred_element_type=_F32)


def _mlp_kernel(ne_ref, ctx_ref, wo_ref, w1_ref, w2_ref, hid_ref):
    h0 = ne_ref[...] + jnp.dot(ctx_ref[...], wo_ref[...],
                               preferred_element_type=_F32)
    h1 = jax.nn.gelu(jnp.dot(h0, w1_ref[...], preferred_element_type=_F32))
    hid_ref[...] = h0 + jnp.dot(h1, w2_ref[...], preferred_element_type=_F32)


def _ce_kernel(hid_ref, wlm_ref, tgt_ref, w_ref,
               loss_ref, acc_ref,
               m_s, s_s, tl_s, bv_s, bi_s):
    j = pl.program_id(0)

    @pl.when(j == 0)
    def _():
        m_s[...] = jnp.full((QL, 1), -1e30, _F32)
        s_s[...] = jnp.zeros((QL, 1), _F32)
        tl_s[...] = jnp.zeros((QL, 1), _F32)
        bv_s[...] = jnp.full((QL, 1), -1e30, _F32)
        bi_s[...] = jnp.zeros((QL, 1), _I32)

    logits = jnp.dot(hid_ref[...], wlm_ref[...], preferred_element_type=_F32)
    gcol = lax.broadcasted_iota(_I32, (QL, VT), 1) + j * VT
    t = tgt_ref[...]

    tmax = jnp.max(logits, axis=1, keepdims=True)
    mnew = jnp.maximum(m_s[...], tmax)
    srow = jnp.sum(jnp.exp(logits - mnew), axis=1, keepdims=True)
    s_s[...] = s_s[...] * jnp.exp(m_s[...] - mnew) + srow
    m_s[...] = mnew

    tl_s[...] += jnp.sum(jnp.where(gcol == t, logits, _F32(0.0)),
                         axis=1, keepdims=True)

    targ = jnp.min(jnp.where(logits == tmax, gcol, _I32(2 ** 30)),
                   axis=1, keepdims=True)
    upd = tmax > bv_s[...]
    bi_s[...] = jnp.where(upd, targ, bi_s[...])
    bv_s[...] = jnp.maximum(bv_s[...], tmax)

    @pl.when(j == NVT - 1)
    def _():
        wv = w_ref[...]
        lpt = m_s[...] + jnp.log(s_s[...]) - tl_s[...]
        loss_ref[0, 0] = jnp.sum(lpt * wv) / (jnp.sum(wv) + _F32(1e-6))
        sel = wv > _F32(0.5)
        corr = jnp.sum(jnp.where(sel & (bi_s[...] == t), _F32(1.0), _F32(0.0)))
        cnt = jnp.sum(jnp.where(sel, _F32(1.0), _F32(0.0)))
        acc_ref[0, 0] = corr / jnp.maximum(cnt, _F32(1.0))


def _sample_anchors_fast(lm):
    max_anchor = SEQ - BS
    valid = lm[:max_anchor + 1] > 0.5
    valid_count = valid.sum()
    rv = jax.random.uniform(jax.random.key(42), (1, max_anchor + 1))[0]
    rv = jnp.where(valid, rv, 2.0)
    idxs = jnp.arange(max_anchor + 1)
    masked_idx = jnp.where(valid, idxs, SEQ + 1)
    _, sel = lax.top_k(-rv, NA)
    anchors = jnp.sort(masked_idx[sel])
    keep = jnp.arange(NA) < jnp.minimum(valid_count, NA)
    anchors = jnp.where(keep, anchors, 0).astype(_I32)
    return anchors, keep


def kernel(input_ids, hidden_states, loss_mask, embed_table,
           Wq, Wk, Wv, Wo, W1, W2, Wlm):
    ids = input_ids[0].astype(_I32)
    lm = loss_mask[0].astype(_F32)
    hs = hidden_states[0]

    anchors, keep = _sample_anchors_fast(lm)
    keep_i = keep.astype(_I32)

    labels = anchors[:, None] + jnp.arange(BS, dtype=_I32)[None, :]
    draft_pos = labels.reshape(QL, 1).astype(_F32)
    ctx_pos_i = jnp.maximum(anchors - 1, 0)
    kv_pos = jnp.concatenate([ctx_pos_i.astype(_F32)[:, None], draft_pos],
                             axis=0)

    kvin, tgt, wlm_g = _sc_gather_stage(anchors, keep_i, ids, lm,
                                        embed_table, hs)

    wq_r = _rot_cols(Wq)
    wk_r = _rot_cols(Wk)

    TK = 128
    qf, kp, vp = pl.pallas_call(
        _qkv_proj_kernel,
        grid=(KL // TK,),
        in_specs=[
            pl.BlockSpec((TK, D), lambda i: (i, 0)),
            pl.BlockSpec((TK, 1), lambda i: (i, 0)),
            pl.BlockSpec((D, D), lambda i: (0, 0)),
            pl.BlockSpec((D, D), lambda i: (0, 0)),
            pl.BlockSpec((D, D), lambda i: (0, 0)),
            pl.BlockSpec((D, D), lambda i: (0, 0)),
            pl.BlockSpec((D, D), lambda i: (0, 0)),
        ],
        out_specs=[
            pl.BlockSpec((TK, D), lambda i: (jnp.maximum(i - 1, 0), 0)),
            pl.BlockSpec((TK, D), lambda i: (i, 0)),
            pl.BlockSpec((TK, D), lambda i: (i, 0)),
        ],
        out_shape=[
            jax.ShapeDtypeStruct((QL, D), _F32),
            jax.ShapeDtypeStruct((KL, D), _F32),
            jax.ShapeDtypeStruct((KL, D), _F32),
        ],
    )(kvin, kv_pos, Wq, wq_r, Wk, wk_r, Wv)

    q_r = qf.reshape(QL * H, DH)
    kc = kp[:NA].reshape(NA, 1, D)
    kd = kp[NA:].reshape(NA, BS, D)
    k_r = jnp.concatenate([kc, kd], axis=1).reshape(NA * KB * H, DH)
    vc = vp[:NA].reshape(NA, 1, D)
    vd = vp[NA:].reshape(NA, BS, D)
    v_r = jnp.concatenate([vc, vd], axis=1).reshape(NA * KB * H, DH)

    mask_hh = jnp.asarray(_MASK_HH)
    g_c = jnp.asarray(_G_COLLAPSE)
    gt_c = jnp.asarray(_GT_EXPAND)

    GSTEP = _ATT_BLOCKS_PER_STEP
    ctx_r = pl.pallas_call(
        _attention_kernel,
        grid=(NA // GSTEP,),
        in_specs=[
            pl.BlockSpec((GSTEP * 256, DH), lambda i: (i, 0)),
            pl.BlockSpec((GSTEP * 272, DH), lambda i: (i, 0)),
            pl.BlockSpec((GSTEP * 272, DH), lambda i: (i, 0)),
            pl.BlockSpec((BS * H, KB * H), lambda i: (0, 0)),
            pl.BlockSpec((KB * H, 32), lambda i: (0, 0)),
            pl.BlockSpec((32, KB * H), lambda i: (0, 0)),
        ],
        out_specs=pl.BlockSpec((GSTEP * 256, DH), lambda i: (i, 0)),
        out_shape=jax.ShapeDtypeStruct((QL * H, DH), _F32),
    )(q_r, k_r, v_r, mask_hh, g_c, gt_c)
    ctx_p = ctx_r.reshape(QL, D)

    TM = 128
    hid = pl.pallas_call(
        _mlp_kernel,
        grid=(QL // TM,),
        in_specs=[
            pl.BlockSpec((TM, D), lambda i: (i + 1, 0)),
            pl.BlockSpec((TM, D), lambda i: (i, 0)),
            pl.BlockSpec((D, D), lambda i: (0, 0)),
            pl.BlockSpec((D, DFF), lambda i: (0, 0)),
            pl.BlockSpec((DFF, D), lambda i: (0, 0)),
        ],
        out_specs=pl.BlockSpec((TM, D), lambda i: (i, 0)),
        out_shape=jax.ShapeDtypeStruct((QL, D), _F32),
    )(kvin, ctx_p, Wo, W1, W2)

    valid_label = (labels < SEQ).astype(_F32)
    wgt = (keep.astype(_F32)[:, None] * valid_label
           * (jnp.arange(BS) > 0).astype(_F32)[None, :]).reshape(QL)
    wgt = (wgt * wlm_g).reshape(QL, 1)
    tgt2 = tgt.reshape(QL, 1)

    loss2, acc2 = pl.pallas_call(
        _ce_kernel,
        grid=(NVT,),
        in_specs=[
            pl.BlockSpec((QL, D), lambda j: (0, 0)),
            pl.BlockSpec((D, VT), lambda j: (0, j)),
            pl.BlockSpec((QL, 1), lambda j: (0, 0)),
            pl.BlockSpec((QL, 1), lambda j: (0, 0)),
        ],
        out_specs=[
            pl.BlockSpec(memory_space=pltpu.SMEM),
            pl.BlockSpec(memory_space=pltpu.SMEM),
        ],
        out_shape=[
            jax.ShapeDtypeStruct((1, 1), _F32),
            jax.ShapeDtypeStruct((1, 1), _F32),
        ],
        scratch_shapes=[
            pltpu.VMEM((QL, 1), _F32),
            pltpu.VMEM((QL, 1), _F32),
            pltpu.VMEM((QL, 1), _F32),
            pltpu.VMEM((QL, 1), _F32),
            pltpu.VMEM((QL, 1), _I32),
        ],
    )(hid, Wlm, tgt2, wgt)

    return loss2[0, 0], acc2[0, 0]


# speedup vs baseline: 1.9139x; 1.1427x over previous
import functools

import numpy as np
import jax
import jax.numpy as jnp
from jax import lax
from jax.experimental import pallas as pl
from jax.experimental.pallas import tpu as pltpu
from jax.experimental.pallas import tpu_sc as plsc

SEQ = 4096
D = 1024
H = 16
DH = 64
HALF = DH // 2
VOCAB = 32000
BS = 16
NA = 128
DFF = 2048
QL = NA * BS
KL = NA + QL
KB = 1 + BS
VT = 1280
NVT = VOCAB // VT

_F32 = jnp.float32
_I32 = jnp.int32


def _np_mask_const():
    r = np.arange(BS * H)[:, None]
    c = np.arange(KB * H)[None, :]
    return (r % H == c % H).astype(np.float32)


def _np_g_const():
    c = np.arange(KB * H)[:, None]
    e = np.arange(32)[None, :]
    return (c // H == e).astype(np.float32)


_MASK_HH = _np_mask_const()
_G_COLLAPSE = _np_g_const()
_GT_EXPAND = _G_COLLAPSE.T.copy()


def _rot_cols(w):
    w3 = w.reshape(D, H, 2, HALF)
    return jnp.stack([-w3[:, :, 1], w3[:, :, 0]], axis=2).reshape(D, D)


def _sc_gather_stage(anchors, keep_i, ids, lm, embed_table, hs):
    mesh = plsc.VectorSubcoreMesh(core_axis_name="c", subcore_axis_name="s")

    @functools.partial(
        pl.kernel,
        mesh=mesh,
        compiler_params=pltpu.CompilerParams(needs_layout_passes=False),
        out_type=[
            jax.ShapeDtypeStruct((KL, D), _F32),
            jax.ShapeDtypeStruct((QL,), _I32),
            jax.ShapeDtypeStruct((QL,), _F32),
        ],
        scratch_types=[
            pltpu.VMEM((NA,), _I32),
            pltpu.VMEM((NA,), _I32),
            pltpu.VMEM((SEQ,), _I32),
            pltpu.VMEM((SEQ,), _F32),
            pltpu.VMEM((4 * BS,), _I32),
            pltpu.VMEM((16,), _I32),
            pltpu.VMEM((4 * BS, D), _F32),
            pltpu.VMEM((16, D), _F32),
            pltpu.VMEM((4 * BS,), _I32),
            pltpu.VMEM((4 * BS,), _F32),
            pltpu.SemaphoreType.DMA,
            pltpu.SemaphoreType.DMA,
        ],
    )
    def sc_kernel(anchors_hbm, keep_hbm, ids_hbm, lm_hbm, embed_hbm, hs_hbm,
                  kvin_hbm, tgt_hbm, wlm_hbm,
                  anchors_v, keep_v, ids_v, lm_v, eidx, cidx,
                  ne_rows, hs_rows, tvec, wvec, sem, sem2):
        wid = lax.axis_index("s") * 2 + lax.axis_index("c")
        lanes = lax.iota(_I32, 16)

        c1 = pltpu.async_copy(anchors_hbm, anchors_v, sem)
        c2 = pltpu.async_copy(keep_hbm, keep_v, sem)
        c3 = pltpu.async_copy(ids_hbm, ids_v, sem)
        c4 = pltpu.async_copy(lm_hbm, lm_v, sem)
        c1.wait()
        c2.wait()
        c3.wait()
        c4.wait()

        b0 = wid * 4
        am = lanes < 4
        aidx = jnp.minimum(b0 + lanes, NA - 1)
        a_v = plsc.load_gather(anchors_v, [aidx])
        k_v = plsc.load_gather(keep_v, [aidx])
        tok_v = plsc.load_gather(ids_v, [jnp.clip(a_v, 0, SEQ - 1)])
        tok_v = jnp.where((k_v > 0) & am, tok_v, 0)
        for j in range(4):
            eidx[pl.ds(16 * j, 16)] = jnp.zeros((16,), _I32)
        plsc.store_scatter(eidx, [lanes * 16], tok_v, mask=am)
        eg = pltpu.async_copy(embed_hbm.at[eidx], ne_rows, sem2)

        for j in range(4):
            bb = jnp.full((16,), b0 + j, _I32)
            a_b = plsc.load_gather(anchors_v, [bb])
            lidx = jnp.clip(a_b + lanes, 0, SEQ - 1)
            tvec[pl.ds(16 * j, 16)] = plsc.load_gather(ids_v, [lidx])
            wvec[pl.ds(16 * j, 16)] = plsc.load_gather(lm_v, [lidx])
        t1 = pltpu.async_copy(tvec, tgt_hbm.at[pl.ds(64 * wid, 64)], sem)
        t2 = pltpu.async_copy(wvec, wlm_hbm.at[pl.ds(64 * wid, 64)], sem)

        eg.wait()
        ne_out = pltpu.async_copy(
            ne_rows, kvin_hbm.at[pl.ds(NA + 64 * wid, 64)], sem2)

        @pl.when(wid < 8)
        def _():
            a16 = anchors_v[pl.ds(16 * wid, 16)]
            cidx[...] = jnp.maximum(a16 - 1, 0)
            pltpu.async_copy(hs_hbm.at[cidx], hs_rows, sem).wait()
            pltpu.sync_copy(hs_rows, kvin_hbm.at[pl.ds(16 * wid, 16)])

        t1.wait()
        t2.wait()
        ne_out.wait()

    return sc_kernel(anchors, keep_i, ids, lm, embed_table, hs)


def _rope_mul(a, b, pos):
    t, _ = a.shape
    col = lax.broadcasted_iota(_I32, (t, D), 1)
    freq = jnp.exp((col % HALF).astype(_F32) * _F32(-np.log(10000.0) / HALF))
    ang = pos * freq
    return a * jnp.cos(ang) + b * jnp.sin(ang)


def _qkv_proj_kernel(x_ref, pos_ref, wq_ref, wqr_ref, wk_ref, wkr_ref, wv_ref,
                     q_ref, k_ref, v_ref):
    x = x_ref[...]
    pos = pos_ref[...]
    aq = jnp.dot(x, wq_ref[...], preferred_element_type=_F32)
    bq = jnp.dot(x, wqr_ref[...], preferred_element_type=_F32)
    q_ref[...] = _rope_mul(aq, bq, pos)
    ak = jnp.dot(x, wk_ref[...], preferred_element_type=_F32)
    bk = jnp.dot(x, wkr_ref[...], preferred_element_type=_F32)
    k_ref[...] = _rope_mul(ak, bk, pos)
    v_ref[...] = jnp.dot(x, wv_ref[...], preferred_element_type=_F32)


_ATT_BLOCKS_PER_STEP = 8


def _attention_kernel(q_ref, kc_ref, kd_ref, vc_ref, vd_ref, m_ref, o_ref):
    m = m_ref[...] > _F32(0.5)
    scale = _F32(1.0 / np.sqrt(DH))
    for b in range(_ATT_BLOCKS_PER_STEP):
        qb = q_ref[256 * b:256 * (b + 1), :]
        kb = jnp.concatenate([kc_ref[16 * b:16 * (b + 1), :],
                              kd_ref[256 * b:256 * (b + 1), :]], axis=0)
        vb = jnp.concatenate([vc_ref[16 * b:16 * (b + 1), :],
                              vd_ref[256 * b:256 * (b + 1), :]], axis=0)
        s = lax.dot_general(qb, kb, (((1,), (1,)), ((), ())),
                            preferred_element_type=_F32) * scale
        mx = jnp.max(jnp.where(m, s, _F32(-1e30)), axis=1, keepdims=True)
        ex = jnp.where(m, jnp.exp(s - mx), _F32(0.0))
        pn = ex / jnp.sum(ex, axis=1, keepdims=True)
        o_ref[256 * b:256 * (b + 1), :] = jnp.dot(
            pn, vb, preferred_element_type=_F32)


def _mlp_kernel(ne_ref, ctx_ref, wo_ref, w1_ref, w2_ref, hid_ref):
    h0 = ne_ref[...] + jnp.dot(ctx_ref[...], wo_ref[...],
                               preferred_element_type=_F32)
    h1 = jax.nn.gelu(jnp.dot(h0, w1_ref[...], preferred_element_type=_F32))
    hid_ref[...] = h0 + jnp.dot(h1, w2_ref[...], preferred_element_type=_F32)


def _ce_kernel(hid_ref, wlm_ref, tgt_ref, w_ref,
               loss_ref, acc_ref,
               m_s, s_s, tl_s, bv_s, bi_s):
    j = pl.program_id(0)

    @pl.when(j == 0)
    def _():
        m_s[...] = jnp.full((QL, 1), -1e30, _F32)
        s_s[...] = jnp.zeros((QL, 1), _F32)
        tl_s[...] = jnp.zeros((QL, 1), _F32)
        bv_s[...] = jnp.full((QL, 1), -1e30, _F32)
        bi_s[...] = jnp.zeros((QL, 1), _I32)

    logits = jnp.dot(hid_ref[...], wlm_ref[...], preferred_element_type=_F32)
    gcol = lax.broadcasted_iota(_I32, (QL, VT), 1) + j * VT
    t = tgt_ref[...]

    tmax = jnp.max(logits, axis=1, keepdims=True)
    mnew = jnp.maximum(m_s[...], tmax)
    srow = jnp.sum(jnp.exp(logits - mnew), axis=1, keepdims=True)
    s_s[...] = s_s[...] * jnp.exp(m_s[...] - mnew) + srow
    m_s[...] = mnew

    tl_s[...] += jnp.sum(jnp.where(gcol == t, logits, _F32(0.0)),
                         axis=1, keepdims=True)

    targ = jnp.min(jnp.where(logits == tmax, gcol, _I32(2 ** 30)),
                   axis=1, keepdims=True)
    upd = tmax > bv_s[...]
    bi_s[...] = jnp.where(upd, targ, bi_s[...])
    bv_s[...] = jnp.maximum(bv_s[...], tmax)

    @pl.when(j == NVT - 1)
    def _():
        wv = w_ref[...]
        lpt = m_s[...] + jnp.log(s_s[...]) - tl_s[...]
        loss_ref[0, 0] = jnp.sum(lpt * wv) / (jnp.sum(wv) + _F32(1e-6))
        sel = wv > _F32(0.5)
        corr = jnp.sum(jnp.where(sel & (bi_s[...] == t), _F32(1.0), _F32(0.0)))
        cnt = jnp.sum(jnp.where(sel, _F32(1.0), _F32(0.0)))
        acc_ref[0, 0] = corr / jnp.maximum(cnt, _F32(1.0))


def _sample_anchors_fast(lm):
    max_anchor = SEQ - BS
    valid = lm[:max_anchor + 1] > 0.5
    valid_count = valid.sum()
    rv = jax.random.uniform(jax.random.key(42), (1, max_anchor + 1))[0]
    rv = jnp.where(valid, rv, 2.0)
    idxs = jnp.arange(max_anchor + 1)
    masked_idx = jnp.where(valid, idxs, SEQ + 1)
    _, sel = lax.top_k(-rv, NA)
    anchors = jnp.sort(masked_idx[sel])
    keep = jnp.arange(NA) < jnp.minimum(valid_count, NA)
    anchors = jnp.where(keep, anchors, 0).astype(_I32)
    return anchors, keep


def kernel(input_ids, hidden_states, loss_mask, embed_table,
           Wq, Wk, Wv, Wo, W1, W2, Wlm):
    ids = input_ids[0].astype(_I32)
    lm = loss_mask[0].astype(_F32)
    hs = hidden_states[0]

    anchors, keep = _sample_anchors_fast(lm)
    keep_i = keep.astype(_I32)

    labels = anchors[:, None] + jnp.arange(BS, dtype=_I32)[None, :]
    draft_pos = labels.reshape(QL, 1).astype(_F32)
    ctx_pos_i = jnp.maximum(anchors - 1, 0)
    kv_pos = jnp.concatenate([ctx_pos_i.astype(_F32)[:, None], draft_pos],
                             axis=0)

    kvin, tgt, wlm_g = _sc_gather_stage(anchors, keep_i, ids, lm,
                                        embed_table, hs)

    wq_r = _rot_cols(Wq)
    wk_r = _rot_cols(Wk)

    TK = 128
    qf, kp, vp = pl.pallas_call(
        _qkv_proj_kernel,
        grid=(KL // TK,),
        in_specs=[
            pl.BlockSpec((TK, D), lambda i: (i, 0)),
            pl.BlockSpec((TK, 1), lambda i: (i, 0)),
            pl.BlockSpec((D, D), lambda i: (0, 0)),
            pl.BlockSpec((D, D), lambda i: (0, 0)),
            pl.BlockSpec((D, D), lambda i: (0, 0)),
            pl.BlockSpec((D, D), lambda i: (0, 0)),
            pl.BlockSpec((D, D), lambda i: (0, 0)),
        ],
        out_specs=[
            pl.BlockSpec((TK, D), lambda i: (jnp.maximum(i - 1, 0), 0)),
            pl.BlockSpec((TK, D), lambda i: (i, 0)),
            pl.BlockSpec((TK, D), lambda i: (i, 0)),
        ],
        out_shape=[
            jax.ShapeDtypeStruct((QL, D), _F32),
            jax.ShapeDtypeStruct((KL, D), _F32),
            jax.ShapeDtypeStruct((KL, D), _F32),
        ],
    )(kvin, kv_pos, Wq, wq_r, Wk, wk_r, Wv)

    q_r = qf.reshape(QL * H, DH)
    k_flat = kp.reshape(KL * H, DH)
    v_flat = vp.reshape(KL * H, DH)

    mask_hh = jnp.asarray(_MASK_HH)

    GSTEP = _ATT_BLOCKS_PER_STEP
    ctx_r = pl.pallas_call(
        _attention_kernel,
        grid=(NA // GSTEP,),
        in_specs=[
            pl.BlockSpec((GSTEP * 256, DH), lambda i: (i, 0)),
            pl.BlockSpec((GSTEP * 16, DH), lambda i: (i, 0)),
            pl.BlockSpec((GSTEP * 256, DH), lambda i: (i + 1, 0)),
            pl.BlockSpec((GSTEP * 16, DH), lambda i: (i, 0)),
            pl.BlockSpec((GSTEP * 256, DH), lambda i: (i + 1, 0)),
            pl.BlockSpec((BS * H, KB * H), lambda i: (0, 0)),
        ],
        out_specs=pl.BlockSpec((GSTEP * 256, DH), lambda i: (i, 0)),
        out_shape=jax.ShapeDtypeStruct((QL * H, DH), _F32),
    )(q_r, k_flat, k_flat, v_flat, v_flat, mask_hh)
    ctx_p = ctx_r.reshape(QL, D)

    TM = 128
    hid = pl.pallas_call(
        _mlp_kernel,
        grid=(QL // TM,),
        in_specs=[
            pl.BlockSpec((TM, D), lambda i: (i + 1, 0)),
            pl.BlockSpec((TM, D), lambda i: (i, 0)),
            pl.BlockSpec((D, D), lambda i: (0, 0)),
            pl.BlockSpec((D, DFF), lambda i: (0, 0)),
            pl.BlockSpec((DFF, D), lambda i: (0, 0)),
        ],
        out_specs=pl.BlockSpec((TM, D), lambda i: (i, 0)),
        out_shape=jax.ShapeDtypeStruct((QL, D), _F32),
    )(kvin, ctx_p, Wo, W1, W2)

    valid_label = (labels < SEQ).astype(_F32)
    wgt = (keep.astype(_F32)[:, None] * valid_label
           * (jnp.arange(BS) > 0).astype(_F32)[None, :]).reshape(QL)
    wgt = (wgt * wlm_g).reshape(QL, 1)
    tgt2 = tgt.reshape(QL, 1)

    loss2, acc2 = pl.pallas_call(
        _ce_kernel,
        grid=(NVT,),
        in_specs=[
            pl.BlockSpec((QL, D), lambda j: (0, 0)),
            pl.BlockSpec((D, VT), lambda j: (0, j)),
            pl.BlockSpec((QL, 1), lambda j: (0, 0)),
            pl.BlockSpec((QL, 1), lambda j: (0, 0)),
        ],
        out_specs=[
            pl.BlockSpec(memory_space=pltpu.SMEM),
            pl.BlockSpec(memory_space=pltpu.SMEM),
        ],
        out_shape=[
            jax.ShapeDtypeStruct((1, 1), _F32),
            jax.ShapeDtypeStruct((1, 1), _F32),
        ],
        scratch_shapes=[
            pltpu.VMEM((QL, 1), _F32),
            pltpu.VMEM((QL, 1), _F32),
            pltpu.VMEM((QL, 1), _F32),
            pltpu.VMEM((QL, 1), _F32),
            pltpu.VMEM((QL, 1), _I32),
        ],
    )(hid, Wlm, tgt2, wgt)

    return loss2[0, 0], acc2[0, 0]


# speedup vs baseline: 2.1483x; 1.1225x over previous
import functools

import numpy as np
import jax
import jax.numpy as jnp
from jax import lax
from jax.experimental import pallas as pl
from jax.experimental.pallas import tpu as pltpu
from jax.experimental.pallas import tpu_sc as plsc

SEQ = 4096
D = 1024
H = 16
DH = 64
HALF = DH // 2
VOCAB = 32000
BS = 16
NA = 128
DFF = 2048
QL = NA * BS
KL = NA + QL
KB = 1 + BS
SP = 2 * NA + 8
VT = 1280
NVT = VOCAB // VT

_F32 = jnp.float32
_I32 = jnp.int32


def _np_mask_const():
    r = np.arange(BS * H)[:, None]
    c = np.arange(KB * H)[None, :]
    return (r % H == c % H).astype(np.float32)


def _np_upsample_const():
    r = np.arange(2 * NA)[:, None]
    b = np.arange(8)[None, :]
    return (r == BS * b).astype(np.float32)


_MASK_HH = _np_mask_const()
_UPS = _np_upsample_const()


def _rot_cols(w):
    w3 = w.reshape(D, H, 2, HALF)
    return jnp.stack([-w3[:, :, 1], w3[:, :, 0]], axis=2).reshape(D, D)


def _sc_gather_stage(anchors, keep_i, ids, lm, embed_table, hs):
    mesh = plsc.VectorSubcoreMesh(core_axis_name="c", subcore_axis_name="s")

    @functools.partial(
        pl.kernel,
        mesh=mesh,
        compiler_params=pltpu.CompilerParams(needs_layout_passes=False),
        out_type=[
            jax.ShapeDtypeStruct((2 * NA, D), _F32),
            jax.ShapeDtypeStruct((QL,), _I32),
            jax.ShapeDtypeStruct((QL,), _F32),
        ],
        scratch_types=[
            pltpu.VMEM((NA,), _I32),
            pltpu.VMEM((NA,), _I32),
            pltpu.VMEM((SEQ,), _I32),
            pltpu.VMEM((SEQ,), _F32),
            pltpu.VMEM((16,), _I32),
            pltpu.VMEM((16, D), _F32),
            pltpu.VMEM((4 * BS,), _I32),
            pltpu.VMEM((4 * BS,), _F32),
            pltpu.SemaphoreType.DMA,
            pltpu.SemaphoreType.DMA,
        ],
    )
    def sc_kernel(anchors_hbm, keep_hbm, ids_hbm, lm_hbm, embed_hbm, hs_hbm,
                  xsp_hbm, tgt_hbm, wlm_hbm,
                  anchors_v, keep_v, ids_v, lm_v, cidx,
                  rows_v, tvec, wvec, sem, sem2):
        wid = lax.axis_index("s") * 2 + lax.axis_index("c")
        lanes = lax.iota(_I32, 16)

        c1 = pltpu.async_copy(anchors_hbm, anchors_v, sem)
        c2 = pltpu.async_copy(keep_hbm, keep_v, sem)
        c3 = pltpu.async_copy(ids_hbm, ids_v, sem)
        c4 = pltpu.async_copy(lm_hbm, lm_v, sem)
        c1.wait()
        c2.wait()
        c3.wait()
        c4.wait()

        @pl.when(wid < 8)
        def _():
            a16 = anchors_v[pl.ds(16 * wid, 16)]
            cidx[...] = jnp.maximum(a16 - 1, 0)
            pltpu.async_copy(hs_hbm.at[cidx], rows_v, sem2).wait()
            pltpu.sync_copy(rows_v, xsp_hbm.at[pl.ds(16 * wid, 16)])

        @pl.when((wid >= 8) & (wid < 16))
        def _():
            w2 = wid - 8
            a16 = anchors_v[pl.ds(16 * w2, 16)]
            k16 = keep_v[pl.ds(16 * w2, 16)]
            tok = plsc.load_gather(ids_v, [jnp.clip(a16, 0, SEQ - 1)])
            cidx[...] = jnp.where(k16 > 0, tok, 0)
            pltpu.async_copy(embed_hbm.at[cidx], rows_v, sem2).wait()
            pltpu.sync_copy(rows_v, xsp_hbm.at[pl.ds(NA + 16 * w2, 16)])

        b0 = wid * 4
        for j in range(4):
            bb = jnp.full((16,), b0 + j, _I32)
            a_b = plsc.load_gather(anchors_v, [bb])
            lidx = jnp.clip(a_b + lanes, 0, SEQ - 1)
            tvec[pl.ds(16 * j, 16)] = plsc.load_gather(ids_v, [lidx])
            wvec[pl.ds(16 * j, 16)] = plsc.load_gather(lm_v, [lidx])
        t1 = pltpu.async_copy(tvec, tgt_hbm.at[pl.ds(64 * wid, 64)], sem)
        t2 = pltpu.async_copy(wvec, wlm_hbm.at[pl.ds(64 * wid, 64)], sem)
        t1.wait()
        t2.wait()

    return sc_kernel(anchors, keep_i, ids, lm, embed_table, hs)


def _rope_mul(a, b, pos):
    t, _ = a.shape
    col = lax.broadcasted_iota(_I32, (t, D), 1)
    freq = jnp.exp((col % HALF).astype(_F32) * _F32(-np.log(10000.0) / HALF))
    ang = pos * freq
    return a * jnp.cos(ang) + b * jnp.sin(ang)


def _special_proj_kernel(x_ref, pos_ref, wq_ref, wqr_ref, wk_ref, wkr_ref,
                         wv_ref, kx_ref, vx_ref, qx_ref, bqx_ref, bkx_ref):
    x = x_ref[...]
    ak = jnp.dot(x, wk_ref[...], preferred_element_type=_F32)
    bk = jnp.dot(x, wkr_ref[...], preferred_element_type=_F32)
    kx_ref[...] = _rope_mul(ak, bk, pos_ref[...])
    bkx_ref[...] = bk
    vx_ref[...] = jnp.dot(x, wv_ref[...], preferred_element_type=_F32)
    qx_ref[...] = jnp.dot(x, wq_ref[...], preferred_element_type=_F32)
    bqx_ref[...] = jnp.dot(x, wqr_ref[...], preferred_element_type=_F32)


def _assemble_kernel(qa_ref, q0_ref, bqa_ref, bq0_ref, ka_ref, k0_ref,
                     bka_ref, bk0_ref, va_ref, v0_ref, u_ref, pos_ref,
                     q_ref, k_ref, v_ref):
    m0 = (lax.broadcasted_iota(_I32, (NA, D), 0) % BS) == 0
    u = u_ref[...]

    def asm(anc_ref, base_ref):
        up = jnp.dot(u, anc_ref[...], preferred_element_type=_F32)
        base = jnp.broadcast_to(base_ref[0:1, :], (NA, D))
        return jnp.where(m0, up, base)

    pos = pos_ref[...]
    q_ref[...] = _rope_mul(asm(qa_ref, q0_ref), asm(bqa_ref, bq0_ref), pos)
    k_ref[...] = _rope_mul(asm(ka_ref, k0_ref), asm(bka_ref, bk0_ref), pos)
    v_ref[...] = asm(va_ref, v0_ref)


_ATT_BLOCKS_PER_STEP = 8


def _attention_kernel(q_ref, kc_ref, kd_ref, vc_ref, vd_ref, m_ref, o_ref):
    m = m_ref[...] > _F32(0.5)
    scale = _F32(1.0 / np.sqrt(DH))
    for b in range(_ATT_BLOCKS_PER_STEP):
        qb = q_ref[256 * b:256 * (b + 1), :]
        kb = jnp.concatenate([kc_ref[16 * b:16 * (b + 1), :],
                              kd_ref[256 * b:256 * (b + 1), :]], axis=0)
        vb = jnp.concatenate([vc_ref[16 * b:16 * (b + 1), :],
                              vd_ref[256 * b:256 * (b + 1), :]], axis=0)
        s = lax.dot_general(qb, kb, (((1,), (1,)), ((), ())),
                            preferred_element_type=_F32) * scale
        mx = jnp.max(jnp.where(m, s, _F32(-1e30)), axis=1, keepdims=True)
        ex = jnp.where(m, jnp.exp(s - mx), _F32(0.0))
        pn = ex / jnp.sum(ex, axis=1, keepdims=True)
        o_ref[256 * b:256 * (b + 1), :] = jnp.dot(
            pn, vb, preferred_element_type=_F32)


def _mlp_kernel(ea_ref, e0_ref, u_ref, ctx_ref, wo_ref, w1_ref, w2_ref,
                hid_ref):
    m0 = (lax.broadcasted_iota(_I32, (NA, D), 0) % BS) == 0
    up = jnp.dot(u_ref[...], ea_ref[...], preferred_element_type=_F32)
    ne = jnp.where(m0, up, jnp.broadcast_to(e0_ref[0:1, :], (NA, D)))
    h0 = ne + jnp.dot(ctx_ref[...], wo_ref[...], preferred_element_type=_F32)
    h1 = jax.nn.gelu(jnp.dot(h0, w1_ref[...], preferred_element_type=_F32))
    hid_ref[...] = h0 + jnp.dot(h1, w2_ref[...], preferred_element_type=_F32)


def _ce_kernel(hid_ref, wlm_ref, tgt_ref, w_ref,
               loss_ref, acc_ref,
               m_s, s_s, tl_s, bv_s, bi_s):
    j = pl.program_id(0)

    @pl.when(j == 0)
    def _():
        m_s[...] = jnp.full((QL, 1), -1e30, _F32)
        s_s[...] = jnp.zeros((QL, 1), _F32)
        tl_s[...] = jnp.zeros((QL, 1), _F32)
        bv_s[...] = jnp.full((QL, 1), -1e30, _F32)
        bi_s[...] = jnp.zeros((QL, 1), _I32)

    logits = jnp.dot(hid_ref[...], wlm_ref[...], preferred_element_type=_F32)
    gcol = lax.broadcasted_iota(_I32, (QL, VT), 1) + j * VT
    t = tgt_ref[...]

    tmax = jnp.max(logits, axis=1, keepdims=True)
    mnew = jnp.maximum(m_s[...], tmax)
    srow = jnp.sum(jnp.exp(logits - mnew), axis=1, keepdims=True)
    s_s[...] = s_s[...] * jnp.exp(m_s[...] - mnew) + srow
    m_s[...] = mnew

    tl_s[...] += jnp.sum(jnp.where(gcol == t, logits, _F32(0.0)),
                         axis=1, keepdims=True)

    targ = jnp.min(jnp.where(logits == tmax, gcol, _I32(2 ** 30)),
                   axis=1, keepdims=True)
    upd = tmax > bv_s[...]
    bi_s[...] = jnp.where(upd, targ, bi_s[...])
    bv_s[...] = jnp.maximum(bv_s[...], tmax)

    @pl.when(j == NVT - 1)
    def _():
        wv = w_ref[...]
        lpt = m_s[...] + jnp.log(s_s[...]) - tl_s[...]
        loss_ref[0, 0] = jnp.sum(lpt * wv) / (jnp.sum(wv) + _F32(1e-6))
        sel = wv > _F32(0.5)
        corr = jnp.sum(jnp.where(sel & (bi_s[...] == t), _F32(1.0), _F32(0.0)))
        cnt = jnp.sum(jnp.where(sel, _F32(1.0), _F32(0.0)))
        acc_ref[0, 0] = corr / jnp.maximum(cnt, _F32(1.0))


def _sample_anchors_fast(lm):
    max_anchor = SEQ - BS
    valid = lm[:max_anchor + 1] > 0.5
    valid_count = valid.sum()
    rv = jax.random.uniform(jax.random.key(42), (1, max_anchor + 1))[0]
    rv = jnp.where(valid, rv, 2.0)
    idxs = jnp.arange(max_anchor + 1)
    masked_idx = jnp.where(valid, idxs, SEQ + 1)
    _, sel = lax.top_k(-rv, NA)
    anchors = jnp.sort(masked_idx[sel])
    keep = jnp.arange(NA) < jnp.minimum(valid_count, NA)
    anchors = jnp.where(keep, anchors, 0).astype(_I32)
    return anchors, keep


def kernel(input_ids, hidden_states, loss_mask, embed_table,
           Wq, Wk, Wv, Wo, W1, W2, Wlm):
    ids = input_ids[0].astype(_I32)
    lm = loss_mask[0].astype(_F32)
    hs = hidden_states[0]

    anchors, keep = _sample_anchors_fast(lm)
    keep_i = keep.astype(_I32)

    labels = anchors[:, None] + jnp.arange(BS, dtype=_I32)[None, :]
    draft_pos = labels.reshape(QL, 1).astype(_F32)
    ctx_pos = jnp.maximum(anchors - 1, 0).astype(_F32)[:, None]

    xsp, tgt, wlm_g = _sc_gather_stage(anchors, keep_i, ids, lm,
                                       embed_table, hs)

    wq_r = _rot_cols(Wq)
    wk_r = _rot_cols(Wk)

    e0p = jnp.pad(embed_table[0:1], ((0, 7), (0, 0)))
    xall = jnp.concatenate([xsp, e0p], axis=0)
    pos_sp = jnp.concatenate([ctx_pos, jnp.zeros((SP - NA, 1), _F32)], axis=0)

    kx, vx, qx, bqx, bkx = pl.pallas_call(
        _special_proj_kernel,
        grid=(1,),
        in_specs=[
            pl.BlockSpec((SP, D), lambda i: (0, 0)),
            pl.BlockSpec((SP, 1), lambda i: (0, 0)),
            pl.BlockSpec((D, D), lambda i: (0, 0)),
            pl.BlockSpec((D, D), lambda i: (0, 0)),
            pl.BlockSpec((D, D), lambda i: (0, 0)),
            pl.BlockSpec((D, D), lambda i: (0, 0)),
            pl.BlockSpec((D, D), lambda i: (0, 0)),
        ],
        out_specs=[pl.BlockSpec((SP, D), lambda i: (0, 0))] * 5,
        out_shape=[jax.ShapeDtypeStruct((SP, D), _F32)] * 5,
    )(xall, pos_sp, Wq, wq_r, Wk, wk_r, Wv)

    ups = jnp.asarray(_UPS[:NA, :])
    anc = lambda i: (NA // 8 + i, 0)
    bas = lambda i: (SP // 8 - 1, 0)
    qd, kd, vd = pl.pallas_call(
        _assemble_kernel,
        grid=(BS,),
        in_specs=[
            pl.BlockSpec((8, D), anc), pl.BlockSpec((8, D), bas),
            pl.BlockSpec((8, D), anc), pl.BlockSpec((8, D), bas),
            pl.BlockSpec((8, D), anc), pl.BlockSpec((8, D), bas),
            pl.BlockSpec((8, D), anc), pl.BlockSpec((8, D), bas),
            pl.BlockSpec((8, D), anc), pl.BlockSpec((8, D), bas),
            pl.BlockSpec((NA, 8), lambda i: (0, 0)),
            pl.BlockSpec((NA, 1), lambda i: (i, 0)),
        ],
        out_specs=[pl.BlockSpec((NA, D), lambda i: (i, 0))] * 3,
        out_shape=[jax.ShapeDtypeStruct((QL, D), _F32)] * 3,
    )(qx, qx, bqx, bqx, kx, kx, bkx, bkx, vx, vx, ups, draft_pos)

    q_r = qd.reshape(QL * H, DH)
    kd_flat = kd.reshape(QL * H, DH)
    vd_flat = vd.reshape(QL * H, DH)
    kc_flat = kx.reshape(SP * H, DH)
    vc_flat = vx.reshape(SP * H, DH)

    mask_hh = jnp.asarray(_MASK_HH)

    GSTEP = _ATT_BLOCKS_PER_STEP
    ctx_r = pl.pallas_call(
        _attention_kernel,
        grid=(NA // GSTEP,),
        in_specs=[
            pl.BlockSpec((GSTEP * 256, DH), lambda i: (i, 0)),
            pl.BlockSpec((GSTEP * 16, DH), lambda i: (i, 0)),
            pl.BlockSpec((GSTEP * 256, DH), lambda i: (i, 0)),
            pl.BlockSpec((GSTEP * 16, DH), lambda i: (i, 0)),
            pl.BlockSpec((GSTEP * 256, DH), lambda i: (i, 0)),
            pl.BlockSpec((BS * H, KB * H), lambda i: (0, 0)),
        ],
        out_specs=pl.BlockSpec((GSTEP * 256, DH), lambda i: (i, 0)),
        out_shape=jax.ShapeDtypeStruct((QL * H, DH), _F32),
    )(q_r, kc_flat, kd_flat, vc_flat, vd_flat, mask_hh)
    ctx_p = ctx_r.reshape(QL, D)

    hid = pl.pallas_call(
        _mlp_kernel,
        grid=(BS,),
        in_specs=[
            pl.BlockSpec((8, D), anc),
            pl.BlockSpec((8, D), bas),
            pl.BlockSpec((NA, 8), lambda i: (0, 0)),
            pl.BlockSpec((NA, D), lambda i: (i, 0)),
            pl.BlockSpec((D, D), lambda i: (0, 0)),
            pl.BlockSpec((D, DFF), lambda i: (0, 0)),
            pl.BlockSpec((DFF, D), lambda i: (0, 0)),
        ],
        out_specs=pl.BlockSpec((NA, D), lambda i: (i, 0)),
        out_shape=jax.ShapeDtypeStruct((QL, D), _F32),
    )(xall, xall, ups, ctx_p, Wo, W1, W2)

    valid_label = (labels < SEQ).astype(_F32)
    wgt = (keep.astype(_F32)[:, None] * valid_label
           * (jnp.arange(BS) > 0).astype(_F32)[None, :]).reshape(QL)
    wgt = (wgt * wlm_g).reshape(QL, 1)
    tgt2 = tgt.reshape(QL, 1)

    loss2, acc2 = pl.pallas_call(
        _ce_kernel,
        grid=(NVT,),
        in_specs=[
            pl.BlockSpec((QL, D), lambda j: (0, 0)),
            pl.BlockSpec((D, VT), lambda j: (0, j)),
            pl.BlockSpec((QL, 1), lambda j: (0, 0)),
            pl.BlockSpec((QL, 1), lambda j: (0, 0)),
        ],
        out_specs=[
            pl.BlockSpec(memory_space=pltpu.SMEM),
            pl.BlockSpec(memory_space=pltpu.SMEM),
        ],
        out_shape=[
            jax.ShapeDtypeStruct((1, 1), _F32),
            jax.ShapeDtypeStruct((1, 1), _F32),
        ],
        scratch_shapes=[
            pltpu.VMEM((QL, 1), _F32),
            pltpu.VMEM((QL, 1), _F32),
            pltpu.VMEM((QL, 1), _F32),
            pltpu.VMEM((QL, 1), _F32),
            pltpu.VMEM((QL, 1), _I32),
        ],
    )(hid, Wlm, tgt2, wgt)

    return loss2[0, 0], acc2[0, 0]


# speedup vs baseline: 2.1970x; 1.0226x over previous
import functools

import numpy as np
import jax
import jax.numpy as jnp
from jax import lax
from jax.experimental import pallas as pl
from jax.experimental.pallas import tpu as pltpu
from jax.experimental.pallas import tpu_sc as plsc

SEQ = 4096
D = 1024
H = 16
DH = 64
HALF = DH // 2
VOCAB = 32000
BS = 16
NA = 128
DFF = 2048
QL = NA * BS
QLC = NA * (BS - 1)
KL = NA + QL
KB = 1 + BS
SP = 2 * NA + 8
VT = 1280
NVT = VOCAB // VT

_F32 = jnp.float32
_I32 = jnp.int32


def _np_mask_const():
    r = np.arange(BS * H)[:, None]
    c = np.arange(KB * H)[None, :]
    return (r % H == c % H).astype(np.float32)


def _np_upsample_const():
    r = np.arange(2 * NA)[:, None]
    b = np.arange(8)[None, :]
    return (r == BS * b).astype(np.float32)


def _np_compact_const():
    r = np.arange(120)[:, None]
    c = np.arange(NA)[None, :]
    return (c == BS * (r // (BS - 1)) + 1 + r % (BS - 1)).astype(np.float32)


_MASK_HH = _np_mask_const()
_UPS = _np_upsample_const()
_CSEL = _np_compact_const()


def _rot_cols(w):
    w3 = w.reshape(D, H, 2, HALF)
    return jnp.stack([-w3[:, :, 1], w3[:, :, 0]], axis=2).reshape(D, D)


def _sc_gather_stage(anchors, keep_i, ids, lm, embed_table, hs):
    mesh = plsc.VectorSubcoreMesh(core_axis_name="c", subcore_axis_name="s")

    @functools.partial(
        pl.kernel,
        mesh=mesh,
        compiler_params=pltpu.CompilerParams(needs_layout_passes=False),
        out_type=[
            jax.ShapeDtypeStruct((2 * NA, D), _F32),
            jax.ShapeDtypeStruct((QL,), _I32),
            jax.ShapeDtypeStruct((QL,), _F32),
        ],
        scratch_types=[
            pltpu.VMEM((NA,), _I32),
            pltpu.VMEM((NA,), _I32),
            pltpu.VMEM((SEQ,), _I32),
            pltpu.VMEM((SEQ,), _F32),
            pltpu.VMEM((16,), _I32),
            pltpu.VMEM((16, D), _F32),
            pltpu.VMEM((4 * BS,), _I32),
            pltpu.VMEM((4 * BS,), _F32),
            pltpu.SemaphoreType.DMA,
            pltpu.SemaphoreType.DMA,
        ],
    )
    def sc_kernel(anchors_hbm, keep_hbm, ids_hbm, lm_hbm, embed_hbm, hs_hbm,
                  xsp_hbm, tgt_hbm, wlm_hbm,
                  anchors_v, keep_v, ids_v, lm_v, cidx,
                  rows_v, tvec, wvec, sem, sem2):
        wid = lax.axis_index("s") * 2 + lax.axis_index("c")
        lanes = lax.iota(_I32, 16)

        c1 = pltpu.async_copy(anchors_hbm, anchors_v, sem)
        c2 = pltpu.async_copy(keep_hbm, keep_v, sem)
        c3 = pltpu.async_copy(ids_hbm, ids_v, sem)
        c4 = pltpu.async_copy(lm_hbm, lm_v, sem)
        c1.wait()
        c2.wait()
        c3.wait()
        c4.wait()

        @pl.when(wid < 8)
        def _():
            a16 = anchors_v[pl.ds(16 * wid, 16)]
            cidx[...] = jnp.maximum(a16 - 1, 0)
            pltpu.async_copy(hs_hbm.at[cidx], rows_v, sem2).wait()
            pltpu.sync_copy(rows_v, xsp_hbm.at[pl.ds(16 * wid, 16)])

        @pl.when((wid >= 8) & (wid < 16))
        def _():
            w2 = wid - 8
            a16 = anchors_v[pl.ds(16 * w2, 16)]
            k16 = keep_v[pl.ds(16 * w2, 16)]
            tok = plsc.load_gather(ids_v, [jnp.clip(a16, 0, SEQ - 1)])
            cidx[...] = jnp.where(k16 > 0, tok, 0)
            pltpu.async_copy(embed_hbm.at[cidx], rows_v, sem2).wait()
            pltpu.sync_copy(rows_v, xsp_hbm.at[pl.ds(NA + 16 * w2, 16)])

        b0 = wid * 4
        for j in range(4):
            bb = jnp.full((16,), b0 + j, _I32)
            a_b = plsc.load_gather(anchors_v, [bb])
            lidx = jnp.clip(a_b + lanes, 0, SEQ - 1)
            tvec[pl.ds(16 * j, 16)] = plsc.load_gather(ids_v, [lidx])
            wvec[pl.ds(16 * j, 16)] = plsc.load_gather(lm_v, [lidx])
        t1 = pltpu.async_copy(tvec, tgt_hbm.at[pl.ds(64 * wid, 64)], sem)
        t2 = pltpu.async_copy(wvec, wlm_hbm.at[pl.ds(64 * wid, 64)], sem)
        t1.wait()
        t2.wait()

    return sc_kernel(anchors, keep_i, ids, lm, embed_table, hs)


def _rope_mul(a, b, pos):
    t, _ = a.shape
    col = lax.broadcasted_iota(_I32, (t, D), 1)
    freq = jnp.exp((col % HALF).astype(_F32) * _F32(-np.log(10000.0) / HALF))
    ang = pos * freq
    return a * jnp.cos(ang) + b * jnp.sin(ang)


def _special_proj_kernel(x_ref, pos_ref, wq_ref, wqr_ref, wk_ref, wkr_ref,
                         wv_ref, kx_ref, vx_ref, qx_ref, bqx_ref, bkx_ref):
    x = x_ref[...]
    ak = jnp.dot(x, wk_ref[...], preferred_element_type=_F32)
    bk = jnp.dot(x, wkr_ref[...], preferred_element_type=_F32)
    kx_ref[...] = _rope_mul(ak, bk, pos_ref[...])
    bkx_ref[...] = bk
    vx_ref[...] = jnp.dot(x, wv_ref[...], preferred_element_type=_F32)
    qx_ref[...] = jnp.dot(x, wq_ref[...], preferred_element_type=_F32)
    bqx_ref[...] = jnp.dot(x, wqr_ref[...], preferred_element_type=_F32)


def _assemble_kernel(qa_ref, q0_ref, bqa_ref, bq0_ref, ka_ref, k0_ref,
                     bka_ref, bk0_ref, va_ref, v0_ref, u_ref, pos_ref,
                     q_ref, k_ref, v_ref):
    m0 = (lax.broadcasted_iota(_I32, (NA, D), 0) % BS) == 0
    u = u_ref[...]

    def asm(anc_ref, base_ref):
        up = jnp.dot(u, anc_ref[...], preferred_element_type=_F32)
        base = jnp.broadcast_to(base_ref[0:1, :], (NA, D))
        return jnp.where(m0, up, base)

    col = lax.broadcasted_iota(_I32, (NA, D), 1)
    freq = jnp.exp((col % HALF).astype(_F32) * _F32(-np.log(10000.0) / HALF))
    ang = pos_ref[...] * freq
    c, s = jnp.cos(ang), jnp.sin(ang)
    q_ref[...] = asm(qa_ref, q0_ref) * c + asm(bqa_ref, bq0_ref) * s
    k_ref[...] = asm(ka_ref, k0_ref) * c + asm(bka_ref, bk0_ref) * s
    v_ref[...] = asm(va_ref, v0_ref)


_ATT_BLOCKS_PER_STEP = 8


def _attention_kernel(q_ref, kc_ref, kd_ref, vc_ref, vd_ref, m_ref, o_ref):
    m = m_ref[...] > _F32(0.5)
    scale = _F32(1.0 / np.sqrt(DH))
    for b in range(_ATT_BLOCKS_PER_STEP):
        qb = q_ref[256 * b:256 * (b + 1), :]
        kb = jnp.concatenate([kc_ref[16 * b:16 * (b + 1), :],
                              kd_ref[256 * b:256 * (b + 1), :]], axis=0)
        vb = jnp.concatenate([vc_ref[16 * b:16 * (b + 1), :],
                              vd_ref[256 * b:256 * (b + 1), :]], axis=0)
        s = lax.dot_general(qb, kb, (((1,), (1,)), ((), ())),
                            preferred_element_type=_F32) * scale
        mx = jnp.max(jnp.where(m, s, _F32(-1e30)), axis=1, keepdims=True)
        ex = jnp.where(m, jnp.exp(s - mx), _F32(0.0))
        pn = ex / jnp.sum(ex, axis=1, keepdims=True)
        o_ref[256 * b:256 * (b + 1), :] = jnp.dot(
            pn, vb, preferred_element_type=_F32)


def _mlp_kernel(ea_ref, e0_ref, u_ref, csel_ref, ctx_ref, wo_ref, w1_ref,
                w2_ref, hid_ref):
    m0 = (lax.broadcasted_iota(_I32, (NA, D), 0) % BS) == 0
    up = jnp.dot(u_ref[...], ea_ref[...], preferred_element_type=_F32)
    ne = jnp.where(m0, up, jnp.broadcast_to(e0_ref[0:1, :], (NA, D)))
    h0 = ne + jnp.dot(ctx_ref[...], wo_ref[...], preferred_element_type=_F32)
    h1 = jax.nn.gelu(jnp.dot(h0, w1_ref[...], preferred_element_type=_F32))
    hid = h0 + jnp.dot(h1, w2_ref[...], preferred_element_type=_F32)
    hid_ref[...] = jnp.dot(csel_ref[...], hid, preferred_element_type=_F32)


def _ce_kernel(hid_ref, wlm_ref, tgt_ref, w_ref,
               loss_ref, acc_ref,
               m_s, s_s, tl_s, bv_s, bi_s):
    j = pl.program_id(0)

    @pl.when(j == 0)
    def _():
        m_s[...] = jnp.full((QLC, 1), -1e30, _F32)
        s_s[...] = jnp.zeros((QLC, 1), _F32)
        tl_s[...] = jnp.zeros((QLC, 1), _F32)
        bv_s[...] = jnp.full((QLC, 1), -1e30, _F32)
        bi_s[...] = jnp.zeros((QLC, 1), _I32)

    logits = jnp.dot(hid_ref[...], wlm_ref[...], preferred_element_type=_F32)
    gcol = lax.broadcasted_iota(_I32, (QLC, VT), 1) + j * VT
    t = tgt_ref[...]

    tmax = jnp.max(logits, axis=1, keepdims=True)
    mnew = jnp.maximum(m_s[...], tmax)
    srow = jnp.sum(jnp.exp(logits - mnew), axis=1, keepdims=True)
    s_s[...] = s_s[...] * jnp.exp(m_s[...] - mnew) + srow
    m_s[...] = mnew

    tl_s[...] += jnp.sum(jnp.where(gcol == t, logits, _F32(0.0)),
                         axis=1, keepdims=True)

    targ = jnp.min(jnp.where(logits == tmax, gcol, _I32(2 ** 30)),
                   axis=1, keepdims=True)
    upd = tmax > bv_s[...]
    bi_s[...] = jnp.where(upd, targ, bi_s[...])
    bv_s[...] = jnp.maximum(bv_s[...], tmax)

    @pl.when(j == NVT - 1)
    def _():
        wv = w_ref[...]
        lpt = m_s[...] + jnp.log(s_s[...]) - tl_s[...]
        loss_ref[0, 0] = jnp.sum(lpt * wv) / (jnp.sum(wv) + _F32(1e-6))
        sel = wv > _F32(0.5)
        corr = jnp.sum(jnp.where(sel & (bi_s[...] == t), _F32(1.0), _F32(0.0)))
        cnt = jnp.sum(jnp.where(sel, _F32(1.0), _F32(0.0)))
        acc_ref[0, 0] = corr / jnp.maximum(cnt, _F32(1.0))


def _sample_anchors_fast(lm):
    max_anchor = SEQ - BS
    valid = lm[:max_anchor + 1] > 0.5
    valid_count = valid.sum()
    rv = jax.random.uniform(jax.random.key(42), (1, max_anchor + 1))[0]
    rv = jnp.where(valid, rv, 2.0)
    idxs = jnp.arange(max_anchor + 1)
    masked_idx = jnp.where(valid, idxs, SEQ + 1)
    _, sel = lax.top_k(-rv, NA)
    anchors = jnp.sort(masked_idx[sel])
    keep = jnp.arange(NA) < jnp.minimum(valid_count, NA)
    anchors = jnp.where(keep, anchors, 0).astype(_I32)
    return anchors, keep


def kernel(input_ids, hidden_states, loss_mask, embed_table,
           Wq, Wk, Wv, Wo, W1, W2, Wlm):
    ids = input_ids[0].astype(_I32)
    lm = loss_mask[0].astype(_F32)
    hs = hidden_states[0]

    anchors, keep = _sample_anchors_fast(lm)
    keep_i = keep.astype(_I32)

    labels = anchors[:, None] + jnp.arange(BS, dtype=_I32)[None, :]
    draft_pos = labels.reshape(QL, 1).astype(_F32)
    ctx_pos = jnp.maximum(anchors - 1, 0).astype(_F32)[:, None]

    xsp, tgt, wlm_g = _sc_gather_stage(anchors, keep_i, ids, lm,
                                       embed_table, hs)

    wq_r = _rot_cols(Wq)
    wk_r = _rot_cols(Wk)

    e0p = jnp.pad(embed_table[0:1], ((0, 7), (0, 0)))
    xall = jnp.concatenate([xsp, e0p], axis=0)
    pos_sp = jnp.concatenate([ctx_pos, jnp.zeros((SP - NA, 1), _F32)], axis=0)

    kx, vx, qx, bqx, bkx = pl.pallas_call(
        _special_proj_kernel,
        grid=(1,),
        in_specs=[
            pl.BlockSpec((SP, D), lambda i: (0, 0)),
            pl.BlockSpec((SP, 1), lambda i: (0, 0)),
            pl.BlockSpec((D, D), lambda i: (0, 0)),
            pl.BlockSpec((D, D), lambda i: (0, 0)),
            pl.BlockSpec((D, D), lambda i: (0, 0)),
            pl.BlockSpec((D, D), lambda i: (0, 0)),
            pl.BlockSpec((D, D), lambda i: (0, 0)),
        ],
        out_specs=[pl.BlockSpec((SP, D), lambda i: (0, 0))] * 5,
        out_shape=[jax.ShapeDtypeStruct((SP, D), _F32)] * 5,
    )(xall, pos_sp, Wq, wq_r, Wk, wk_r, Wv)

    ups = jnp.asarray(_UPS[:NA, :])
    anc = lambda i: (NA // 8 + i, 0)
    bas = lambda i: (SP // 8 - 1, 0)
    qd, kd, vd = pl.pallas_call(
        _assemble_kernel,
        grid=(BS,),
        in_specs=[
            pl.BlockSpec((8, D), anc), pl.BlockSpec((8, D), bas),
            pl.BlockSpec((8, D), anc), pl.BlockSpec((8, D), bas),
            pl.BlockSpec((8, D), anc), pl.BlockSpec((8, D), bas),
            pl.BlockSpec((8, D), anc), pl.BlockSpec((8, D), bas),
            pl.BlockSpec((8, D), anc), pl.BlockSpec((8, D), bas),
            pl.BlockSpec((NA, 8), lambda i: (0, 0)),
            pl.BlockSpec((NA, 1), lambda i: (i, 0)),
        ],
        out_specs=[pl.BlockSpec((NA, D), lambda i: (i, 0))] * 3,
        out_shape=[jax.ShapeDtypeStruct((QL, D), _F32)] * 3,
    )(qx, qx, bqx, bqx, kx, kx, bkx, bkx, vx, vx, ups, draft_pos)

    q_r = qd.reshape(QL * H, DH)
    kd_flat = kd.reshape(QL * H, DH)
    vd_flat = vd.reshape(QL * H, DH)
    kc_flat = kx.reshape(SP * H, DH)
    vc_flat = vx.reshape(SP * H, DH)

    mask_hh = jnp.asarray(_MASK_HH)

    GSTEP = _ATT_BLOCKS_PER_STEP
    ctx_r = pl.pallas_call(
        _attention_kernel,
        grid=(NA // GSTEP,),
        in_specs=[
            pl.BlockSpec((GSTEP * 256, DH), lambda i: (i, 0)),
            pl.BlockSpec((GSTEP * 16, DH), lambda i: (i, 0)),
            pl.BlockSpec((GSTEP * 256, DH), lambda i: (i, 0)),
            pl.BlockSpec((GSTEP * 16, DH), lambda i: (i, 0)),
            pl.BlockSpec((GSTEP * 256, DH), lambda i: (i, 0)),
            pl.BlockSpec((BS * H, KB * H), lambda i: (0, 0)),
        ],
        out_specs=pl.BlockSpec((GSTEP * 256, DH), lambda i: (i, 0)),
        out_shape=jax.ShapeDtypeStruct((QL * H, DH), _F32),
    )(q_r, kc_flat, kd_flat, vc_flat, vd_flat, mask_hh)
    ctx_p = ctx_r.reshape(QL, D)

    csel = jnp.asarray(_CSEL)
    hid = pl.pallas_call(
        _mlp_kernel,
        grid=(BS,),
        in_specs=[
            pl.BlockSpec((8, D), anc),
            pl.BlockSpec((8, D), bas),
            pl.BlockSpec((NA, 8), lambda i: (0, 0)),
            pl.BlockSpec((8 * (BS - 1), NA), lambda i: (0, 0)),
            pl.BlockSpec((NA, D), lambda i: (i, 0)),
            pl.BlockSpec((D, D), lambda i: (0, 0)),
            pl.BlockSpec((D, DFF), lambda i: (0, 0)),
            pl.BlockSpec((DFF, D), lambda i: (0, 0)),
        ],
        out_specs=pl.BlockSpec((8 * (BS - 1), D), lambda i: (i, 0)),
        out_shape=jax.ShapeDtypeStruct((QLC, D), _F32),
    )(xall, xall, ups, csel, ctx_p, Wo, W1, W2)

    valid_label = (labels < SEQ).astype(_F32)
    wgt = keep.astype(_F32)[:, None] * valid_label * wlm_g.reshape(NA, BS)
    wgt = wgt[:, 1:].reshape(QLC, 1)
    tgt2 = tgt.reshape(NA, BS)[:, 1:].reshape(QLC, 1)

    loss2, acc2 = pl.pallas_call(
        _ce_kernel,
        grid=(NVT,),
        in_specs=[
            pl.BlockSpec((QLC, D), lambda j: (0, 0)),
            pl.BlockSpec((D, VT), lambda j: (0, j)),
            pl.BlockSpec((QLC, 1), lambda j: (0, 0)),
            pl.BlockSpec((QLC, 1), lambda j: (0, 0)),
        ],
        out_specs=[
            pl.BlockSpec(memory_space=pltpu.SMEM),
            pl.BlockSpec(memory_space=pltpu.SMEM),
        ],
        out_shape=[
            jax.ShapeDtypeStruct((1, 1), _F32),
            jax.ShapeDtypeStruct((1, 1), _F32),
        ],
        scratch_shapes=[
            pltpu.VMEM((QLC, 1), _F32),
            pltpu.VMEM((QLC, 1), _F32),
            pltpu.VMEM((QLC, 1), _F32),
            pltpu.VMEM((QLC, 1), _F32),
            pltpu.VMEM((QLC, 1), _I32),
        ],
    )(hid, Wlm, tgt2, wgt)

    return loss2[0, 0], acc2[0, 0]


# speedup vs baseline: 2.2174x; 1.0093x over previous
import functools

import numpy as np
import jax
import jax.numpy as jnp
from jax import lax
from jax.experimental import pallas as pl
from jax.experimental.pallas import tpu as pltpu
from jax.experimental.pallas import tpu_sc as plsc

SEQ = 4096
D = 1024
H = 16
DH = 64
HALF = DH // 2
VOCAB = 32000
BS = 16
NA = 128
DFF = 2048
QL = NA * BS
QLC = NA * (BS - 1)
KL = NA + QL
KB = 1 + BS
SP = 2 * NA + 8
VT = 1280
NVT = VOCAB // VT

_F32 = jnp.float32
_I32 = jnp.int32


def _np_mask_const():
    r = np.arange(BS * H)[:, None]
    c = np.arange(KB * H)[None, :]
    return (r % H == c % H).astype(np.float32)


def _np_upsample_const():
    r = np.arange(2 * NA)[:, None]
    b = np.arange(8)[None, :]
    return (r == BS * b).astype(np.float32)


def _np_asm_const():
    r = np.arange(NA)[:, None]
    c = np.arange(16)[None, :]
    return np.where(c < 8, r == BS * c, (c == 8) & (r % BS != 0)
                    ).astype(np.float32)


def _np_compact_const():
    r = np.arange(120)[:, None]
    c = np.arange(NA)[None, :]
    return (c == BS * (r // (BS - 1)) + 1 + r % (BS - 1)).astype(np.float32)


_MASK_HH = _np_mask_const()
_UPS = _np_upsample_const()
_ASM = _np_asm_const()
_CSEL = _np_compact_const()


def _rot_cols(w):
    w3 = w.reshape(D, H, 2, HALF)
    return jnp.stack([-w3[:, :, 1], w3[:, :, 0]], axis=2).reshape(D, D)


def _sc_gather_stage(anchors, keep_i, ids, lm, embed_table, hs):
    mesh = plsc.VectorSubcoreMesh(core_axis_name="c", subcore_axis_name="s")

    @functools.partial(
        pl.kernel,
        mesh=mesh,
        compiler_params=pltpu.CompilerParams(needs_layout_passes=False),
        out_type=[
            jax.ShapeDtypeStruct((2 * NA, D), _F32),
            jax.ShapeDtypeStruct((QL,), _I32),
            jax.ShapeDtypeStruct((QL,), _F32),
        ],
        scratch_types=[
            pltpu.VMEM((NA,), _I32),
            pltpu.VMEM((NA,), _I32),
            pltpu.VMEM((SEQ,), _I32),
            pltpu.VMEM((SEQ,), _F32),
            pltpu.VMEM((16,), _I32),
            pltpu.VMEM((16, D), _F32),
            pltpu.VMEM((4 * BS,), _I32),
            pltpu.VMEM((4 * BS,), _F32),
            pltpu.SemaphoreType.DMA,
            pltpu.SemaphoreType.DMA,
        ],
    )
    def sc_kernel(anchors_hbm, keep_hbm, ids_hbm, lm_hbm, embed_hbm, hs_hbm,
                  xsp_hbm, tgt_hbm, wlm_hbm,
                  anchors_v, keep_v, ids_v, lm_v, cidx,
                  rows_v, tvec, wvec, sem, sem2):
        wid = lax.axis_index("s") * 2 + lax.axis_index("c")
        lanes = lax.iota(_I32, 16)

        c1 = pltpu.async_copy(anchors_hbm, anchors_v, sem)
        c2 = pltpu.async_copy(keep_hbm, keep_v, sem)
        c3 = pltpu.async_copy(ids_hbm, ids_v, sem)
        c4 = pltpu.async_copy(lm_hbm, lm_v, sem)
        c1.wait()
        c2.wait()
        c3.wait()
        c4.wait()

        @pl.when(wid < 8)
        def _():
            a16 = anchors_v[pl.ds(16 * wid, 16)]
            cidx[...] = jnp.maximum(a16 - 1, 0)
            pltpu.async_copy(hs_hbm.at[cidx], rows_v, sem2).wait()
            pltpu.sync_copy(rows_v, xsp_hbm.at[pl.ds(16 * wid, 16)])

        @pl.when((wid >= 8) & (wid < 16))
        def _():
            w2 = wid - 8
            a16 = anchors_v[pl.ds(16 * w2, 16)]
            k16 = keep_v[pl.ds(16 * w2, 16)]
            tok = plsc.load_gather(ids_v, [jnp.clip(a16, 0, SEQ - 1)])
            cidx[...] = jnp.where(k16 > 0, tok, 0)
            pltpu.async_copy(embed_hbm.at[cidx], rows_v, sem2).wait()
            pltpu.sync_copy(rows_v, xsp_hbm.at[pl.ds(NA + 16 * w2, 16)])

        b0 = wid * 4
        for j in range(4):
            bb = jnp.full((16,), b0 + j, _I32)
            a_b = plsc.load_gather(anchors_v, [bb])
            lidx = jnp.clip(a_b + lanes, 0, SEQ - 1)
            tvec[pl.ds(16 * j, 16)] = plsc.load_gather(ids_v, [lidx])
            wvec[pl.ds(16 * j, 16)] = plsc.load_gather(lm_v, [lidx])
        t1 = pltpu.async_copy(tvec, tgt_hbm.at[pl.ds(64 * wid, 64)], sem)
        t2 = pltpu.async_copy(wvec, wlm_hbm.at[pl.ds(64 * wid, 64)], sem)
        t1.wait()
        t2.wait()

    return sc_kernel(anchors, keep_i, ids, lm, embed_table, hs)


def _rope_mul(a, b, pos):
    t, _ = a.shape
    col = lax.broadcasted_iota(_I32, (t, D), 1)
    freq = jnp.exp((col % HALF).astype(_F32) * _F32(-np.log(10000.0) / HALF))
    ang = pos * freq
    return a * jnp.cos(ang) + b * jnp.sin(ang)


def _special_proj_kernel(x_ref, pos_ref, wq_ref, wqr_ref, wk_ref, wkr_ref,
                         wv_ref, kx_ref, vx_ref, qx_ref, bqx_ref, bkx_ref):
    x = x_ref[...]
    ak = jnp.dot(x, wk_ref[...], preferred_element_type=_F32)
    bk = jnp.dot(x, wkr_ref[...], preferred_element_type=_F32)
    kx_ref[...] = _rope_mul(ak, bk, pos_ref[...])
    bkx_ref[...] = bk
    vx_ref[...] = jnp.dot(x, wv_ref[...], preferred_element_type=_F32)
    qx_ref[...] = jnp.dot(x, wq_ref[...], preferred_element_type=_F32)
    bqx_ref[...] = jnp.dot(x, wqr_ref[...], preferred_element_type=_F32)


def _assemble_kernel(qa_ref, q0_ref, bqa_ref, bq0_ref, ka_ref, k0_ref,
                     bka_ref, bk0_ref, va_ref, v0_ref, u_ref, pos_ref,
                     q_ref, k_ref, v_ref):
    u = u_ref[...]

    def asm(anc_ref, base_ref):
        a = jnp.concatenate([anc_ref[...], base_ref[...]], axis=0)
        return jnp.dot(u, a, preferred_element_type=_F32)

    col = lax.broadcasted_iota(_I32, (NA, D), 1)
    freq = jnp.exp((col % HALF).astype(_F32) * _F32(-np.log(10000.0) / HALF))
    ang = pos_ref[...] * freq
    c, s = jnp.cos(ang), jnp.sin(ang)
    q_ref[...] = asm(qa_ref, q0_ref) * c + asm(bqa_ref, bq0_ref) * s
    k_ref[...] = asm(ka_ref, k0_ref) * c + asm(bka_ref, bk0_ref) * s
    v_ref[...] = asm(va_ref, v0_ref)


_ATT_BLOCKS_PER_STEP = 16


def _attention_kernel(q_ref, kc_ref, kd_ref, vc_ref, vd_ref, m_ref, o_ref):
    m = m_ref[...] > _F32(0.5)
    scale = _F32(1.0 / np.sqrt(DH))
    for b in range(_ATT_BLOCKS_PER_STEP):
        qb = q_ref[256 * b:256 * (b + 1), :]
        kb = jnp.concatenate([kc_ref[16 * b:16 * (b + 1), :],
                              kd_ref[256 * b:256 * (b + 1), :]], axis=0)
        vb = jnp.concatenate([vc_ref[16 * b:16 * (b + 1), :],
                              vd_ref[256 * b:256 * (b + 1), :]], axis=0)
        s = lax.dot_general(qb, kb, (((1,), (1,)), ((), ())),
                            preferred_element_type=_F32) * scale
        mx = jnp.max(jnp.where(m, s, _F32(-1e30)), axis=1, keepdims=True)
        ex = jnp.where(m, jnp.exp(s - mx), _F32(0.0))
        pn = ex / jnp.sum(ex, axis=1, keepdims=True)
        o_ref[256 * b:256 * (b + 1), :] = jnp.dot(
            pn, vb, preferred_element_type=_F32)


def _mlp_kernel(ea_ref, e0_ref, u_ref, csel_ref, ctx_ref, wo_ref, w1_ref,
                w2_ref, hid_ref):
    m0 = (lax.broadcasted_iota(_I32, (NA, D), 0) % BS) == 0
    up = jnp.dot(u_ref[...], ea_ref[...], preferred_element_type=_F32)
    ne = jnp.where(m0, up, jnp.broadcast_to(e0_ref[0:1, :], (NA, D)))
    h0 = ne + jnp.dot(ctx_ref[...], wo_ref[...], preferred_element_type=_F32)
    h1 = jax.nn.gelu(jnp.dot(h0, w1_ref[...], preferred_element_type=_F32))
    hid = h0 + jnp.dot(h1, w2_ref[...], preferred_element_type=_F32)
    hid_ref[...] = jnp.dot(csel_ref[...], hid, preferred_element_type=_F32)


def _ce_kernel(hid_ref, wlm_ref, tgt_ref, w_ref,
               loss_ref, acc_ref,
               m_s, s_s, tl_s, bv_s, bi_s):
    j = pl.program_id(0)

    @pl.when(j == 0)
    def _():
        m_s[...] = jnp.full((QLC, 1), -1e30, _F32)
        s_s[...] = jnp.zeros((QLC, 1), _F32)
        tl_s[...] = jnp.zeros((QLC, 1), _F32)
        bv_s[...] = jnp.full((QLC, 1), -1e30, _F32)
        bi_s[...] = jnp.zeros((QLC, 1), _I32)

    logits = jnp.dot(hid_ref[...], wlm_ref[...], preferred_element_type=_F32)
    gcol = lax.broadcasted_iota(_I32, (QLC, VT), 1) + j * VT
    t = tgt_ref[...]

    tmax = jnp.max(logits, axis=1, keepdims=True)
    mnew = jnp.maximum(m_s[...], tmax)
    srow = jnp.sum(jnp.exp(logits - mnew), axis=1, keepdims=True)
    s_s[...] = s_s[...] * jnp.exp(m_s[...] - mnew) + srow
    m_s[...] = mnew

    tl_s[...] += jnp.sum(jnp.where(gcol == t, logits, _F32(0.0)),
                         axis=1, keepdims=True)

    targ = jnp.min(jnp.where(logits == tmax, gcol, _I32(2 ** 30)),
                   axis=1, keepdims=True)
    upd = tmax > bv_s[...]
    bi_s[...] = jnp.where(upd, targ, bi_s[...])
    bv_s[...] = jnp.maximum(bv_s[...], tmax)

    @pl.when(j == NVT - 1)
    def _():
        wv = w_ref[...]
        lpt = m_s[...] + jnp.log(s_s[...]) - tl_s[...]
        loss_ref[0, 0] = jnp.sum(lpt * wv) / (jnp.sum(wv) + _F32(1e-6))
        sel = wv > _F32(0.5)
        corr = jnp.sum(jnp.where(sel & (bi_s[...] == t), _F32(1.0), _F32(0.0)))
        cnt = jnp.sum(jnp.where(sel, _F32(1.0), _F32(0.0)))
        acc_ref[0, 0] = corr / jnp.maximum(cnt, _F32(1.0))


def _sample_anchors_fast(lm):
    max_anchor = SEQ - BS
    valid = lm[:max_anchor + 1] > 0.5
    valid_count = valid.sum()
    rv = jax.random.uniform(jax.random.key(42), (1, max_anchor + 1))[0]
    rv = jnp.where(valid, rv, 2.0)
    idxs = jnp.arange(max_anchor + 1)
    masked_idx = jnp.where(valid, idxs, SEQ + 1)
    _, sel = lax.top_k(-rv, NA)
    anchors = jnp.sort(masked_idx[sel])
    keep = jnp.arange(NA) < jnp.minimum(valid_count, NA)
    anchors = jnp.where(keep, anchors, 0).astype(_I32)
    return anchors, keep


def kernel(input_ids, hidden_states, loss_mask, embed_table,
           Wq, Wk, Wv, Wo, W1, W2, Wlm):
    ids = input_ids[0].astype(_I32)
    lm = loss_mask[0].astype(_F32)
    hs = hidden_states[0]

    anchors, keep = _sample_anchors_fast(lm)
    keep_i = keep.astype(_I32)

    labels = anchors[:, None] + jnp.arange(BS, dtype=_I32)[None, :]
    draft_pos = labels.reshape(QL, 1).astype(_F32)
    ctx_pos = jnp.maximum(anchors - 1, 0).astype(_F32)[:, None]

    xsp, tgt, wlm_g = _sc_gather_stage(anchors, keep_i, ids, lm,
                                       embed_table, hs)

    wq_r = _rot_cols(Wq)
    wk_r = _rot_cols(Wk)

    e0p = jnp.pad(embed_table[0:1], ((0, 7), (0, 0)))
    xall = jnp.concatenate([xsp, e0p], axis=0)
    pos_sp = jnp.concatenate([ctx_pos, jnp.zeros((SP - NA, 1), _F32)], axis=0)

    kx, vx, qx, bqx, bkx = pl.pallas_call(
        _special_proj_kernel,
        grid=(1,),
        in_specs=[
            pl.BlockSpec((SP, D), lambda i: (0, 0)),
            pl.BlockSpec((SP, 1), lambda i: (0, 0)),
            pl.BlockSpec((D, D), lambda i: (0, 0)),
            pl.BlockSpec((D, D), lambda i: (0, 0)),
            pl.BlockSpec((D, D), lambda i: (0, 0)),
            pl.BlockSpec((D, D), lambda i: (0, 0)),
            pl.BlockSpec((D, D), lambda i: (0, 0)),
        ],
        out_specs=[pl.BlockSpec((SP, D), lambda i: (0, 0))] * 5,
        out_shape=[jax.ShapeDtypeStruct((SP, D), _F32)] * 5,
    )(xall, pos_sp, Wq, wq_r, Wk, wk_r, Wv)

    asm_m = jnp.asarray(_ASM)
    ups = jnp.asarray(_UPS[:NA, :])
    anc = lambda i: (NA // 8 + i, 0)
    bas = lambda i: (SP // 8 - 1, 0)
    qd, kd, vd = pl.pallas_call(
        _assemble_kernel,
        grid=(BS,),
        in_specs=[
            pl.BlockSpec((8, D), anc), pl.BlockSpec((8, D), bas),
            pl.BlockSpec((8, D), anc), pl.BlockSpec((8, D), bas),
            pl.BlockSpec((8, D), anc), pl.BlockSpec((8, D), bas),
            pl.BlockSpec((8, D), anc), pl.BlockSpec((8, D), bas),
            pl.BlockSpec((8, D), anc), pl.BlockSpec((8, D), bas),
            pl.BlockSpec((NA, 16), lambda i: (0, 0)),
            pl.BlockSpec((NA, 1), lambda i: (i, 0)),
        ],
        out_specs=[pl.BlockSpec((NA, D), lambda i: (i, 0))] * 3,
        out_shape=[jax.ShapeDtypeStruct((QL, D), _F32)] * 3,
    )(qx, qx, bqx, bqx, kx, kx, bkx, bkx, vx, vx, asm_m, draft_pos)

    q_r = qd.reshape(QL * H, DH)
    kd_flat = kd.reshape(QL * H, DH)
    vd_flat = vd.reshape(QL * H, DH)
    kc_flat = kx.reshape(SP * H, DH)
    vc_flat = vx.reshape(SP * H, DH)

    mask_hh = jnp.asarray(_MASK_HH)

    GSTEP = _ATT_BLOCKS_PER_STEP
    ctx_r = pl.pallas_call(
        _attention_kernel,
        grid=(NA // GSTEP,),
        in_specs=[
            pl.BlockSpec((GSTEP * 256, DH), lambda i: (i, 0)),
            pl.BlockSpec((GSTEP * 16, DH), lambda i: (i, 0)),
            pl.BlockSpec((GSTEP * 256, DH), lambda i: (i, 0)),
            pl.BlockSpec((GSTEP * 16, DH), lambda i: (i, 0)),
            pl.BlockSpec((GSTEP * 256, DH), lambda i: (i, 0)),
            pl.BlockSpec((BS * H, KB * H), lambda i: (0, 0)),
        ],
        out_specs=pl.BlockSpec((GSTEP * 256, DH), lambda i: (i, 0)),
        out_shape=jax.ShapeDtypeStruct((QL * H, DH), _F32),
    )(q_r, kc_flat, kd_flat, vc_flat, vd_flat, mask_hh)
    ctx_p = ctx_r.reshape(QL, D)

    csel = jnp.asarray(_CSEL)
    hid = pl.pallas_call(
        _mlp_kernel,
        grid=(BS,),
        in_specs=[
            pl.BlockSpec((8, D), anc),
            pl.BlockSpec((8, D), bas),
            pl.BlockSpec((NA, 8), lambda i: (0, 0)),
            pl.BlockSpec((8 * (BS - 1), NA), lambda i: (0, 0)),
            pl.BlockSpec((NA, D), lambda i: (i, 0)),
            pl.BlockSpec((D, D), lambda i: (0, 0)),
            pl.BlockSpec((D, DFF), lambda i: (0, 0)),
            pl.BlockSpec((DFF, D), lambda i: (0, 0)),
        ],
        out_specs=pl.BlockSpec((8 * (BS - 1), D), lambda i: (i, 0)),
        out_shape=jax.ShapeDtypeStruct((QLC, D), _F32),
    )(xall, xall, ups, csel, ctx_p, Wo, W1, W2)

    valid_label = (labels < SEQ).astype(_F32)
    wgt = keep.astype(_F32)[:, None] * valid_label * wlm_g.reshape(NA, BS)
    wgt = wgt[:, 1:].reshape(QLC, 1)
    tgt2 = tgt.reshape(NA, BS)[:, 1:].reshape(QLC, 1)

    loss2, acc2 = pl.pallas_call(
        _ce_kernel,
        grid=(NVT,),
        in_specs=[
            pl.BlockSpec((QLC, D), lambda j: (0, 0)),
            pl.BlockSpec((D, VT), lambda j: (0, j)),
            pl.BlockSpec((QLC, 1), lambda j: (0, 0)),
            pl.BlockSpec((QLC, 1), lambda j: (0, 0)),
        ],
        out_specs=[
            pl.BlockSpec(memory_space=pltpu.SMEM),
            pl.BlockSpec(memory_space=pltpu.SMEM),
        ],
        out_shape=[
            jax.ShapeDtypeStruct((1, 1), _F32),
            jax.ShapeDtypeStruct((1, 1), _F32),
        ],
        scratch_shapes=[
            pltpu.VMEM((QLC, 1), _F32),
            pltpu.VMEM((QLC, 1), _F32),
            pltpu.VMEM((QLC, 1), _F32),
            pltpu.VMEM((QLC, 1), _F32),
            pltpu.VMEM((QLC, 1), _I32),
        ],
    )(hid, Wlm, tgt2, wgt)

    return loss2[0, 0], acc2[0, 0]
